# baseline (speedup 1.0000x reference)
import functools

import jax
import jax.numpy as jnp
from jax import lax
from jax.experimental import pallas as pl
from jax.experimental.pallas import tpu as pltpu
from jax.experimental.pallas import tpu_sc as plsc

LENGTH = 100000
D = 128
B = 16384
MOM = 0.5

NC, NS, LANES = 2, 16, 16
NW = NC * NS
BPW = B // NW
SHARD = 3200
LAST = LENGTH - (NW - 1) * SHARD
CHUNK = 128
LISTCAP = SHARD + CHUNK
CC = 200
NBUF = 3
MAXSTEPS = SHARD // CC


def _mesh():
    return plsc.VectorSubcoreMesh(core_axis_name="c", subcore_axis_name="s")


def _iota16():
    return lax.iota(jnp.int32, 16)


def _sc_gather_dedup(bank, ind, newlab):
    @functools.partial(
        pl.kernel,
        out_type=(
            jax.ShapeDtypeStruct((B, D), jnp.float32),
            jax.ShapeDtypeStruct((NW, LISTCAP), jnp.int32),
            jax.ShapeDtypeStruct((NW, LISTCAP), jnp.int32),
            jax.ShapeDtypeStruct((NW, LISTCAP), jnp.int32),
            jax.ShapeDtypeStruct((NW, 16), jnp.int32),
        ),
        mesh=_mesh(),
        compiler_params=pltpu.CompilerParams(needs_layout_passes=False),
        scratch_types=[
            pltpu.VMEM((B,), jnp.int32),
            pltpu.VMEM((B,), jnp.int32),
            pltpu.VMEM((BPW, D), jnp.float32),
            pltpu.VMEM((SHARD,), jnp.int32),
            pltpu.VMEM((LISTCAP,), jnp.int32),
            pltpu.VMEM((LISTCAP,), jnp.int32),
            pltpu.VMEM((LISTCAP,), jnp.int32),
            pltpu.VMEM((16,), jnp.int32),
            pltpu.SemaphoreType.DMA,
        ],
    )
    def k(bank_hbm, ind_hbm, nlab_hbm,
          old_hbm, posl_hbm, lidxl_hbm, labl_hbm, cnt_hbm,
          ind_v, labf_v, rows_v, win_v, posl_v, lidxl_v, labl_v, cnt_v, gsem):
        wid = lax.axis_index("s") * NC + lax.axis_index("c")
        base = wid * SHARD
        hi = jnp.minimum(base + SHARD, LENGTH)
        iota = _iota16()

        pltpu.sync_copy(ind_hbm, ind_v)
        gdesc = pltpu.make_async_copy(
            bank_hbm.at[ind_v.at[pl.ds(wid * BPW, BPW)]], rows_v, gsem)
        gdesc.start()
        pltpu.sync_copy(nlab_hbm, labf_v)

        def zero_body(t, _):
            win_v[pl.ds(t * 16, 16)] = jnp.zeros((16,), jnp.int32)
            return 0
        lax.fori_loop(0, SHARD // 16, zero_body, 0)

        with jax.named_scope("ph_scan"):
            def scan_body(j, _):
                v = ind_v[pl.ds(j * 16, 16)]
                pos1 = iota + (j * 16 + 1)
                m = jnp.logical_and(v >= base, v < hi)
                lidx = jnp.where(m, v - base, 0)
                plsc.store_scatter(win_v, [lidx], pos1, mask=m)
                return 0
            lax.fori_loop(0, B // 16, scan_body, 0)

        with jax.named_scope("ph_compact"):
            def cmp_body(t, off):
                wv = win_v[pl.ds(t * 16, 16)]
                m = wv > 0
                mi = m.astype(jnp.int32)
                pos = wv - 1
                tgt = off + plsc.cumsum(mi) - mi
                tgt = jnp.where(m, tgt, 0)
                plsc.store_scatter(posl_v, [tgt], pos, mask=m)
                lrow = iota + t * 16
                plsc.store_scatter(lidxl_v, [tgt], lrow, mask=m)
                lv = plsc.load_gather(labf_v, [jnp.where(m, pos, 0)], mask=m)
                plsc.store_scatter(labl_v, [tgt], lv, mask=m)
                return off + jnp.sum(mi)
            nw_cnt = lax.fori_loop(0, SHARD // 16, cmp_body, jnp.int32(0))

        p0 = posl_v[pl.ds(0, 16)][0]
        l0 = lidxl_v[pl.ds(0, 16)][0]

        def pad_body(t, _):
            gi = iota + t * 16
            sel = gi >= nw_cnt
            posl_v[pl.ds(t * 16, 16)] = jnp.where(
                sel, p0, posl_v[pl.ds(t * 16, 16)])
            lidxl_v[pl.ds(t * 16, 16)] = jnp.where(
                sel, l0, lidxl_v[pl.ds(t * 16, 16)])
            return 0
        lax.fori_loop(0, LISTCAP // 16, pad_body, 0)

        cnt_v[pl.ds(0, 16)] = jnp.zeros((16,), jnp.int32) + nw_cnt

        pltpu.sync_copy(posl_v, posl_hbm.at[wid])
        pltpu.sync_copy(lidxl_v, lidxl_hbm.at[wid])
        pltpu.sync_copy(labl_v, labl_hbm.at[wid])
        pltpu.sync_copy(cnt_v, cnt_hbm.at[wid])
        gdesc.wait()
        pltpu.sync_copy(rows_v, old_hbm.at[pl.ds(wid * BPW, BPW)])

    return k(bank, ind, newlab)


def _tc_blend(old, feature):
    RB = 2048

    def body(o_ref, f_ref, out_ref):
        f = f_ref[...]
        o = o_ref[...]
        fn = f / (jnp.sqrt(jnp.sum(f * f, axis=1, keepdims=True)) + 1e-10)
        nw = MOM * o + (1.0 - MOM) * fn
        out_ref[...] = nw / (jnp.sqrt(jnp.sum(nw * nw, axis=1, keepdims=True)) + 1e-10)

    return pl.pallas_call(
        body,
        grid=(B // RB,),
        in_specs=[
            pl.BlockSpec((RB, D), lambda i: (i, 0)),
            pl.BlockSpec((RB, D), lambda i: (i, 0)),
        ],
        out_specs=pl.BlockSpec((RB, D), lambda i: (i, 0)),
        out_shape=jax.ShapeDtypeStruct((B, D), jnp.float32),
    )(old, feature)


def _sc_update(bank, labels, fnew, posl, lidxl, labl, cnts):
    @functools.partial(
        pl.kernel,
        out_type=(
            jax.ShapeDtypeStruct((LENGTH, D), jnp.float32),
            jax.ShapeDtypeStruct((LENGTH,), jnp.int32),
        ),
        mesh=_mesh(),
        compiler_params=pltpu.CompilerParams(needs_layout_passes=False),
        scratch_types=[
            pltpu.VMEM((LISTCAP,), jnp.int32),
            pltpu.VMEM((LISTCAP,), jnp.int32),
            pltpu.VMEM((LISTCAP,), jnp.int32),
            pltpu.VMEM((16,), jnp.int32),
            pltpu.VMEM((SHARD,), jnp.int32),
            pltpu.VMEM((CHUNK,), jnp.int32),
            pltpu.VMEM((CHUNK,), jnp.int32),
            pltpu.VMEM((CHUNK, D), jnp.float32),
            pltpu.VMEM((CC, D), jnp.float32),
            pltpu.VMEM((CC, D), jnp.float32),
            pltpu.VMEM((CC, D), jnp.float32),
            pltpu.SemaphoreType.DMA,
            pltpu.SemaphoreType.DMA,
            pltpu.SemaphoreType.DMA,
            pltpu.SemaphoreType.DMA,
            pltpu.SemaphoreType.DMA,
            pltpu.SemaphoreType.DMA,
            pltpu.SemaphoreType.DMA,
            pltpu.SemaphoreType.DMA,
        ],
    )
    def k(bank_hbm, lab_hbm, fnew_hbm, posl_hbm, lidxl_hbm, labl_hbm, cnt_hbm,
          out_fb, out_lb,
          posl_v, lidxl_v, labl_v, cnt_v, labsh_v, posbuf, bidxbuf, rowbuf,
          cb0, cb1, cb2, rs0, rs1, rs2, ws0, ws1, ws2, gsem, ssem):
        wid = lax.axis_index("s") * NC + lax.axis_index("c")
        base = wid * SHARD
        is_last = wid == NW - 1
        steps = jnp.where(is_last, LAST // CC, MAXSTEPS)
        cbufs, rsems, wsems = (cb0, cb1, cb2), (rs0, rs1, rs2), (ws0, ws1, ws2)
        iota = _iota16()

        def rd_desc(i, b):
            return pltpu.make_async_copy(
                bank_hbm.at[pl.ds(base + i * CC, CC)], cbufs[b], rsems[b])

        def wr_desc(i, b):
            return pltpu.make_async_copy(
                cbufs[b], out_fb.at[pl.ds(base + i * CC, CC)], wsems[b])

        for i in range(NBUF):
            @pl.when(jnp.int32(i) < steps)
            def _(i=i):
                rd_desc(i, i).start()

        pltpu.sync_copy(posl_hbm.at[wid], posl_v)
        pltpu.sync_copy(lidxl_hbm.at[wid], lidxl_v)
        pltpu.sync_copy(labl_hbm.at[wid], labl_v)
        pltpu.sync_copy(cnt_hbm.at[wid], cnt_v)

        @pl.when(jnp.logical_not(is_last))
        def _():
            pltpu.sync_copy(lab_hbm.at[pl.ds(base, SHARD)], labsh_v)

        @pl.when(is_last)
        def _():
            pltpu.sync_copy(lab_hbm.at[pl.ds(base, LAST)],
                            labsh_v.at[pl.ds(0, LAST)])

        nw_cnt = cnt_v[pl.ds(0, 16)][0]

        with jax.named_scope("ph_labels"):
            def lab_body(j, _):
                m = (iota + j * 16) < nw_cnt
                lidx = jnp.where(m, lidxl_v[pl.ds(j * 16, 16)], 0)
                lv = labl_v[pl.ds(j * 16, 16)]
                plsc.store_scatter(labsh_v, [lidx], lv, mask=m)
                return 0
            lax.fori_loop(0, (nw_cnt + 15) // 16, lab_body, 0)

        with jax.named_scope("ph_copy"):
            for i in range(MAXSTEPS):
                b = i % NBUF

                @pl.when(jnp.int32(i) < steps)
                def _(i=i, b=b):
                    if i >= NBUF:
                        wr_desc(i - NBUF, b).wait()
                        rd_desc(i, b).start()
                    rd_desc(i, b).wait()
                    wr_desc(i, b).start()

            for b in range(NBUF):
                @pl.when(jnp.int32(b) < steps)
                def _(b=b):
                    wr_desc(0, b).wait()

        with jax.named_scope("ph_chunks"):
            nchunks = (nw_cnt + CHUNK - 1) // CHUNK

            def chunk_body(i, _):
                c = i * CHUNK
                for kk in range(CHUNK // 16):
                    posbuf[pl.ds(kk * 16, 16)] = posl_v[pl.ds(c + kk * 16, 16)]
                    bidxbuf[pl.ds(kk * 16, 16)] = (
                        lidxl_v[pl.ds(c + kk * 16, 16)] + base)
                pltpu.async_copy(fnew_hbm.at[posbuf], rowbuf, gsem).wait()
                pltpu.async_copy(rowbuf, out_fb.at[bidxbuf], ssem).wait()
                return 0
            lax.fori_loop(0, nchunks, chunk_body, 0)

        @pl.when(jnp.logical_not(is_last))
        def _():
            pltpu.sync_copy(labsh_v, out_lb.at[pl.ds(base, SHARD)])

        @pl.when(is_last)
        def _():
            pltpu.sync_copy(labsh_v.at[pl.ds(0, LAST)],
                            out_lb.at[pl.ds(base, LAST)])

    return k(bank, labels, fnew, posl, lidxl, labl, cnts)


def kernel(feature_bank, label_bank, ind, feature, label):
    ind = ind.astype(jnp.int32)
    label = label.astype(jnp.int32)
    old, posl, lidxl, labl, cnts = _sc_gather_dedup(feature_bank, ind, label)
    fnew = _tc_blend(old, feature)
    return _sc_update(feature_bank, label_bank, fnew, posl, lidxl, labl, cnts)


# speedup vs baseline: 1.0643x; 1.0643x over previous
import functools

import jax
import jax.numpy as jnp
from jax import lax
from jax.experimental import pallas as pl
from jax.experimental.pallas import tpu as pltpu
from jax.experimental.pallas import tpu_sc as plsc

LENGTH = 100000
D = 128
B = 16384
MOM = 0.5

NC, NS, LANES = 2, 16, 16
NW = NC * NS
SHARD = 3200
LAST = LENGTH - (NW - 1) * SHARD
CC = 160
NBUF = 3
MAXSTEPS = SHARD // CC
TPR = CC // 16
FGR = 32
FNROWS = CC + FGR + 8
LISTCAP = SHARD + 256


def _mesh():
    return plsc.VectorSubcoreMesh(core_axis_name="c", subcore_axis_name="s")


def _iota16():
    return lax.iota(jnp.int32, 16)


def _rsqrt(x):
    xi = plsc.bitcast(x, jnp.int32)
    yi = jnp.int32(0x5F3759DF) - lax.shift_right_arithmetic(xi, 1)
    y = plsc.bitcast(yi, jnp.float32)
    for _ in range(3):
        y = y * (1.5 - 0.5 * x * y * y)
    return y


def _sc_dedup(ind, newlab):
    @functools.partial(
        pl.kernel,
        out_type=(
            jax.ShapeDtypeStruct((NW, LISTCAP), jnp.int32),
            jax.ShapeDtypeStruct((NW, LISTCAP), jnp.int32),
            jax.ShapeDtypeStruct((NW, LISTCAP), jnp.int32),
            jax.ShapeDtypeStruct((NW, 48), jnp.int32),
            jax.ShapeDtypeStruct((NW, 16), jnp.int32),
        ),
        mesh=_mesh(),
        compiler_params=pltpu.CompilerParams(needs_layout_passes=False),
        scratch_types=[
            pltpu.VMEM((B,), jnp.int32),
            pltpu.VMEM((B,), jnp.int32),
            pltpu.VMEM((SHARD,), jnp.int32),
            pltpu.VMEM((LISTCAP,), jnp.int32),
            pltpu.VMEM((LISTCAP,), jnp.int32),
            pltpu.VMEM((LISTCAP,), jnp.int32),
            pltpu.VMEM((48,), jnp.int32),
            pltpu.VMEM((16,), jnp.int32),
        ],
    )
    def k(ind_hbm, nlab_hbm,
          posl_hbm, lidxl_hbm, labl_hbm, lob_hbm, cnt_hbm,
          ind_v, labf_v, win_v, posl_v, lidxl_v, labl_v, lobuf_v, cnt_v):
        wid = lax.axis_index("s") * NC + lax.axis_index("c")
        base = wid * SHARD
        hi = jnp.minimum(base + SHARD, LENGTH)
        iota = _iota16()
        lane0 = iota == 0

        pltpu.sync_copy(ind_hbm, ind_v)
        pltpu.sync_copy(nlab_hbm, labf_v)

        def zero_body(t, _):
            win_v[pl.ds(t * 16, 16)] = jnp.zeros((16,), jnp.int32)
            return 0
        lax.fori_loop(0, SHARD // 16, zero_body, 0)

        with jax.named_scope("ph_scan"):
            def scan_body(j, _):
                v = ind_v[pl.ds(j * 16, 16)]
                pos1 = iota + (j * 16 + 1)
                m = jnp.logical_and(v >= base, v < hi)
                lidx = jnp.where(m, v - base, 0)
                plsc.store_scatter(win_v, [lidx], pos1, mask=m)
                return 0
            lax.fori_loop(0, B // 16, scan_body, 0)

        with jax.named_scope("ph_compact"):
            def cmp_body(t, off):
                @pl.when(t % TPR == 0)
                def _():
                    plsc.store_scatter(
                        lobuf_v, [jnp.zeros((16,), jnp.int32) + t // TPR],
                        jnp.zeros((16,), jnp.int32) + off, mask=lane0)
                wv = win_v[pl.ds(t * 16, 16)]
                m = wv > 0
                mi = m.astype(jnp.int32)
                pos = wv - 1
                tgt = off + plsc.cumsum(mi) - mi
                tgt = jnp.where(m, tgt, 0)
                plsc.store_scatter(posl_v, [tgt], pos, mask=m)
                lrow = iota + t * 16
                plsc.store_scatter(lidxl_v, [tgt], lrow, mask=m)
                lv = plsc.load_gather(labf_v, [jnp.where(m, pos, 0)], mask=m)
                plsc.store_scatter(labl_v, [tgt], lv, mask=m)
                return off + jnp.sum(mi)
            nw_cnt = lax.fori_loop(0, SHARD // 16, cmp_body, jnp.int32(0))

        l0v = lobuf_v[pl.ds(0, 16)]
        lobuf_v[pl.ds(0, 16)] = jnp.where(iota >= MAXSTEPS, nw_cnt, l0v)
        l1v = lobuf_v[pl.ds(16, 16)]
        lobuf_v[pl.ds(16, 16)] = jnp.where(iota + 16 >= MAXSTEPS, nw_cnt, l1v)
        lobuf_v[pl.ds(32, 16)] = jnp.zeros((16,), jnp.int32) + nw_cnt

        p0 = posl_v[pl.ds(0, 16)][0]
        q0 = lidxl_v[pl.ds(0, 16)][0]

        def pad_body(t, _):
            gi = iota + t * 16
            sel = gi >= nw_cnt
            posl_v[pl.ds(t * 16, 16)] = jnp.where(
                sel, p0, posl_v[pl.ds(t * 16, 16)])
            lidxl_v[pl.ds(t * 16, 16)] = jnp.where(
                sel, q0, lidxl_v[pl.ds(t * 16, 16)])
            return 0
        lax.fori_loop(0, LISTCAP // 16, pad_body, 0)

        cnt_v[pl.ds(0, 16)] = jnp.zeros((16,), jnp.int32) + nw_cnt

        pltpu.sync_copy(posl_v, posl_hbm.at[wid])
        pltpu.sync_copy(lidxl_v, lidxl_hbm.at[wid])
        pltpu.sync_copy(labl_v, labl_hbm.at[wid])
        pltpu.sync_copy(lobuf_v, lob_hbm.at[wid])
        pltpu.sync_copy(cnt_v, cnt_hbm.at[wid])

    return k(ind, newlab)


def _tc_norm(feature):
    RB = 2048

    def body(f_ref, out_ref):
        f = f_ref[...]
        out_ref[...] = f / (jnp.sqrt(jnp.sum(f * f, axis=1, keepdims=True)) + 1e-10)

    return pl.pallas_call(
        body,
        grid=(B // RB,),
        in_specs=[pl.BlockSpec((RB, D), lambda i: (i, 0))],
        out_specs=pl.BlockSpec((RB, D), lambda i: (i, 0)),
        out_shape=jax.ShapeDtypeStruct((B, D), jnp.float32),
    )(feature)


def _sc_update(bank, labels, fnorm, posl, lidxl, labl, lob, cnts):
    @functools.partial(
        pl.kernel,
        out_type=(
            jax.ShapeDtypeStruct((LENGTH, D), jnp.float32),
            jax.ShapeDtypeStruct((LENGTH,), jnp.int32),
        ),
        mesh=_mesh(),
        compiler_params=pltpu.CompilerParams(needs_layout_passes=False),
        scratch_types=[
            pltpu.VMEM((LISTCAP,), jnp.int32),
            pltpu.VMEM((LISTCAP,), jnp.int32),
            pltpu.VMEM((LISTCAP,), jnp.int32),
            pltpu.VMEM((48,), jnp.int32),
            pltpu.VMEM((16,), jnp.int32),
            pltpu.VMEM((SHARD,), jnp.int32),
            pltpu.VMEM((CC, D), jnp.float32),
            pltpu.VMEM((CC, D), jnp.float32),
            pltpu.VMEM((CC, D), jnp.float32),
            pltpu.VMEM((FNROWS, D), jnp.float32),
            pltpu.VMEM((FNROWS, D), jnp.float32),
            pltpu.SemaphoreType.DMA,
            pltpu.SemaphoreType.DMA,
            pltpu.SemaphoreType.DMA,
            pltpu.SemaphoreType.DMA,
            pltpu.SemaphoreType.DMA,
            pltpu.SemaphoreType.DMA,
            pltpu.SemaphoreType.DMA,
            pltpu.SemaphoreType.DMA,
        ],
    )
    def k(bank_hbm, lab_hbm, fn_hbm, posl_hbm, lidxl_hbm, labl_hbm, lob_hbm,
          cnt_hbm, out_fb, out_lb,
          posl_v, lidxl_v, labl_v, lobuf_v, cnt_v, labsh_v,
          cb0, cb1, cb2, fb0, fb1,
          rs0, rs1, rs2, ws0, ws1, ws2, fs0, fs1):
        wid = lax.axis_index("s") * NC + lax.axis_index("c")
        base = wid * SHARD
        is_last = wid == NW - 1
        steps = jnp.where(is_last, LAST // CC, MAXSTEPS)
        cbufs, rsems, wsems = (cb0, cb1, cb2), (rs0, rs1, rs2), (ws0, ws1, ws2)
        fnbufs, fnsems = (fb0, fb1), (fs0, fs1)
        iota = _iota16()

        def rd_desc(i, b):
            return pltpu.make_async_copy(
                bank_hbm.at[pl.ds(base + i * CC, CC)], cbufs[b], rsems[b])

        def wr_desc(i, b):
            return pltpu.make_async_copy(
                cbufs[b], out_fb.at[pl.ds(base + i * CC, CC)], wsems[b])

        for i in range(NBUF):
            @pl.when(jnp.int32(i) < steps)
            def _(i=i):
                rd_desc(i, i).start()

        pltpu.sync_copy(posl_hbm.at[wid], posl_v)
        pltpu.sync_copy(lidxl_hbm.at[wid], lidxl_v)
        pltpu.sync_copy(labl_hbm.at[wid], labl_v)
        pltpu.sync_copy(lob_hbm.at[wid], lobuf_v)
        pltpu.sync_copy(cnt_hbm.at[wid], cnt_v)

        @pl.when(jnp.logical_not(is_last))
        def _():
            pltpu.sync_copy(lab_hbm.at[pl.ds(base, SHARD)], labsh_v)

        @pl.when(is_last)
        def _():
            pltpu.sync_copy(lab_hbm.at[pl.ds(base, LAST)],
                            labsh_v.at[pl.ds(0, LAST)])

        nw_cnt = cnt_v[pl.ds(0, 16)][0]

        def region_lo(j):
            vb = (j // 16) * 16
            return lobuf_v[pl.ds(vb, 16)][j % 16]

        def region_lo8(j):
            return pl.multiple_of((region_lo(j) // 8) * 8, 8)

        def fn_n(j):
            return (region_lo(j + 1) - region_lo8(j) + FGR - 1) // FGR

        def fn_gather_start(j, f):
            lo8 = region_lo8(j)

            def sbody(g, _):
                off = pl.multiple_of(lo8 + g * FGR, 8)
                pltpu.make_async_copy(
                    fn_hbm.at[posl_v.at[pl.ds(off, FGR)]],
                    fnbufs[f].at[pl.ds(g * FGR, FGR)], fnsems[f]).start()
                return 0
            lax.fori_loop(0, fn_n(j), sbody, 0)

        def fn_gather_wait(j, f):
            lo8 = region_lo8(j)

            def wbody(g, _):
                pltpu.make_async_copy(
                    fn_hbm.at[posl_v.at[pl.ds(lo8, FGR)]],
                    fnbufs[f].at[pl.ds(0, FGR)], fnsems[f]).wait()
                return 0
            lax.fori_loop(0, fn_n(j), wbody, 0)

        with jax.named_scope("ph_labels"):
            def lab_body(j, _):
                m = (iota + j * 16) < nw_cnt
                lidx = jnp.where(m, lidxl_v[pl.ds(j * 16, 16)], 0)
                lv = labl_v[pl.ds(j * 16, 16)]
                plsc.store_scatter(labsh_v, [lidx], lv, mask=m)
                return 0
            lax.fori_loop(0, (nw_cnt + 15) // 16, lab_body, 0)

        fn_gather_start(0, 0)

        def modify_region(i, b, f):
            lo = region_lo(i)
            lo8 = region_lo8(i)
            hie = region_lo(i + 1)
            cbuf = cbufs[b]
            fnbuf = fnbufs[f]

            iot = _iota16()

            def wbody(r, _):
                rsp = jnp.zeros((16,), jnp.int32) + r
                rowsel = plsc.load_gather(lidxl_v, [rsp]) - i * CC
                frsel = jnp.zeros((16,), jnp.int32) + (r - lo8)
                acc = jnp.zeros((16,), jnp.float32)
                nws = []
                for kk in range(D // 16):
                    cols = iot + kk * 16
                    o = plsc.load_gather(cbuf, [rowsel, cols])
                    fn = plsc.load_gather(fnbuf, [frsel, cols])
                    nwk = MOM * o + (1.0 - MOM) * fn
                    nws.append(nwk)
                    acc = acc + nwk * nwk
                s = jnp.sum(acc)
                sv = jnp.zeros((16,), jnp.float32) + s
                sq = sv * _rsqrt(sv)
                inv = 1.0 / (sq + 1e-10)
                for kk in range(D // 16):
                    cols = iot + kk * 16
                    plsc.store_scatter(cbuf, [rowsel, cols], nws[kk] * inv)
                return 0
            lax.fori_loop(lo, hie, wbody, 0)

        with jax.named_scope("ph_ring"):
            for i in range(MAXSTEPS):
                b = i % NBUF
                f = i % 2

                @pl.when(jnp.int32(i) < steps)
                def _(i=i, b=b, f=f):
                    if i >= NBUF:
                        wr_desc(i - NBUF, b).wait()
                        rd_desc(i, b).start()
                    rd_desc(i, b).wait()
                    fn_gather_wait(i, f)
                    if i + 1 < MAXSTEPS:
                        @pl.when(jnp.int32(i + 1) < steps)
                        def _():
                            fn_gather_start(i + 1, 1 - f)
                    modify_region(i, b, f)
                    wr_desc(i, b).start()

            for b in range(NBUF):
                @pl.when(jnp.int32(b) < steps)
                def _(b=b):
                    wr_desc(0, b).wait()

        @pl.when(jnp.logical_not(is_last))
        def _():
            pltpu.sync_copy(labsh_v, out_lb.at[pl.ds(base, SHARD)])

        @pl.when(is_last)
        def _():
            pltpu.sync_copy(labsh_v.at[pl.ds(0, LAST)],
                            out_lb.at[pl.ds(base, LAST)])

    return k(bank, labels, fnorm, posl, lidxl, labl, lob, cnts)


def kernel(feature_bank, label_bank, ind, feature, label):
    ind = ind.astype(jnp.int32)
    label = label.astype(jnp.int32)
    posl, lidxl, labl, lob, cnts = _sc_dedup(ind, label)
    fnorm = _tc_norm(feature)
    return _sc_update(feature_bank, label_bank, fnorm,
                      posl, lidxl, labl, lob, cnts)


# speedup vs baseline: 1.2555x; 1.1796x over previous
import functools

import jax
import jax.numpy as jnp
from jax import lax
from jax.experimental import pallas as pl
from jax.experimental.pallas import tpu as pltpu
from jax.experimental.pallas import tpu_sc as plsc

LENGTH = 100000
D = 128
B = 16384
MOM = 0.5

NC, NS, LANES = 2, 16, 16
NW = NC * NS
SHARD = 3200
LAST = LENGTH - (NW - 1) * SHARD
CC = 160
NBUF = 3
MAXSTEPS = SHARD // CC
TPR = CC // 16
FGR = 32
FNROWS = CC + FGR + 8
LISTCAP = SHARD + 256


def _mesh():
    return plsc.VectorSubcoreMesh(core_axis_name="c", subcore_axis_name="s")


def _iota16():
    return lax.iota(jnp.int32, 16)


def _rsqrt(x):
    xi = plsc.bitcast(x, jnp.int32)
    yi = jnp.int32(0x5F3759DF) - lax.shift_right_arithmetic(xi, 1)
    y = plsc.bitcast(yi, jnp.float32)
    for _ in range(3):
        y = y * (1.5 - 0.5 * x * y * y)
    return y


def _sc_dedup(ind, newlab):
    @functools.partial(
        pl.kernel,
        out_type=(
            jax.ShapeDtypeStruct((NW, LISTCAP), jnp.int32),
            jax.ShapeDtypeStruct((NW, LISTCAP), jnp.int32),
            jax.ShapeDtypeStruct((NW, LISTCAP), jnp.int32),
            jax.ShapeDtypeStruct((NW, 48), jnp.int32),
            jax.ShapeDtypeStruct((NW, 16), jnp.int32),
        ),
        mesh=_mesh(),
        compiler_params=pltpu.CompilerParams(needs_layout_passes=False),
        scratch_types=[
            pltpu.VMEM((B,), jnp.int32),
            pltpu.VMEM((B,), jnp.int32),
            pltpu.VMEM((SHARD,), jnp.int32),
            pltpu.VMEM((LISTCAP,), jnp.int32),
            pltpu.VMEM((LISTCAP,), jnp.int32),
            pltpu.VMEM((LISTCAP,), jnp.int32),
            pltpu.VMEM((48,), jnp.int32),
            pltpu.VMEM((16,), jnp.int32),
            pltpu.SemaphoreType.DMA,
            pltpu.SemaphoreType.DMA,
        ],
    )
    def k(ind_hbm, nlab_hbm,
          posl_hbm, lidxl_hbm, labl_hbm, lob_hbm, cnt_hbm,
          ind_v, labf_v, win_v, posl_v, lidxl_v, labl_v, lobuf_v, cnt_v,
          sem_a, sem_b):
        wid = lax.axis_index("s") * NC + lax.axis_index("c")
        base = wid * SHARD
        hi = jnp.minimum(base + SHARD, LENGTH)
        iota = _iota16()
        lane0 = iota == 0

        d1 = pltpu.make_async_copy(ind_hbm, ind_v, sem_a)
        d2 = pltpu.make_async_copy(nlab_hbm, labf_v, sem_b)
        d1.start()
        d2.start()
        d1.wait()

        def zero_body(t, _):
            win_v[pl.ds(t * 16, 16)] = jnp.zeros((16,), jnp.int32)
            return 0
        lax.fori_loop(0, SHARD // 16, zero_body, 0)

        with jax.named_scope("ph_scan"):
            def scan_body(j, _):
                v = ind_v[pl.ds(j * 16, 16)]
                pos1 = iota + (j * 16 + 1)
                m = jnp.logical_and(v >= base, v < hi)
                lidx = jnp.where(m, v - base, 0)
                plsc.store_scatter(win_v, [lidx], pos1, mask=m)
                return 0
            lax.fori_loop(0, B // 16, scan_body, 0)

        d2.wait()
        with jax.named_scope("ph_compact"):
            def cmp_body(t, off):
                @pl.when(t % TPR == 0)
                def _():
                    plsc.store_scatter(
                        lobuf_v, [jnp.zeros((16,), jnp.int32) + t // TPR],
                        jnp.zeros((16,), jnp.int32) + off, mask=lane0)
                wv = win_v[pl.ds(t * 16, 16)]
                m = wv > 0
                mi = m.astype(jnp.int32)
                pos = wv - 1
                tgt = off + plsc.cumsum(mi) - mi
                tgt = jnp.where(m, tgt, 0)
                plsc.store_scatter(posl_v, [tgt], pos, mask=m)
                lrow = iota + t * 16
                plsc.store_scatter(lidxl_v, [tgt], lrow, mask=m)
                lv = plsc.load_gather(labf_v, [jnp.where(m, pos, 0)], mask=m)
                plsc.store_scatter(labl_v, [tgt], lv, mask=m)
                return off + jnp.sum(mi)
            nw_cnt = lax.fori_loop(0, SHARD // 16, cmp_body, jnp.int32(0))

        l0v = lobuf_v[pl.ds(0, 16)]
        lobuf_v[pl.ds(0, 16)] = jnp.where(iota >= MAXSTEPS, nw_cnt, l0v)
        l1v = lobuf_v[pl.ds(16, 16)]
        lobuf_v[pl.ds(16, 16)] = jnp.where(iota + 16 >= MAXSTEPS, nw_cnt, l1v)
        lobuf_v[pl.ds(32, 16)] = jnp.zeros((16,), jnp.int32) + nw_cnt

        p0 = posl_v[pl.ds(0, 16)][0]
        q0 = lidxl_v[pl.ds(0, 16)][0]

        def pad_body(t, _):
            gi = iota + t * 16
            sel = gi >= nw_cnt
            posl_v[pl.ds(t * 16, 16)] = jnp.where(
                sel, p0, posl_v[pl.ds(t * 16, 16)])
            lidxl_v[pl.ds(t * 16, 16)] = jnp.where(
                sel, q0, lidxl_v[pl.ds(t * 16, 16)])
            return 0
        lax.fori_loop(0, LISTCAP // 16, pad_body, 0)

        cnt_v[pl.ds(0, 16)] = jnp.zeros((16,), jnp.int32) + nw_cnt

        pltpu.sync_copy(posl_v, posl_hbm.at[wid])
        pltpu.sync_copy(lidxl_v, lidxl_hbm.at[wid])
        pltpu.sync_copy(labl_v, labl_hbm.at[wid])
        pltpu.sync_copy(lobuf_v, lob_hbm.at[wid])
        pltpu.sync_copy(cnt_v, cnt_hbm.at[wid])

    return k(ind, newlab)


def _tc_norm(feature):
    RB = 2048

    def body(f_ref, out_ref):
        f = f_ref[...]
        out_ref[...] = f / (jnp.sqrt(jnp.sum(f * f, axis=1, keepdims=True)) + 1e-10)

    return pl.pallas_call(
        body,
        grid=(B // RB,),
        in_specs=[pl.BlockSpec((RB, D), lambda i: (i, 0))],
        out_specs=pl.BlockSpec((RB, D), lambda i: (i, 0)),
        out_shape=jax.ShapeDtypeStruct((B, D), jnp.float32),
    )(feature)


def _sc_update(bank, labels, fnorm, posl, lidxl, labl, lob, cnts):
    @functools.partial(
        pl.kernel,
        out_type=(
            jax.ShapeDtypeStruct((LENGTH, D), jnp.float32),
            jax.ShapeDtypeStruct((LENGTH,), jnp.int32),
        ),
        mesh=_mesh(),
        compiler_params=pltpu.CompilerParams(needs_layout_passes=False),
        scratch_types=[
            pltpu.VMEM((LISTCAP,), jnp.int32),
            pltpu.VMEM((LISTCAP,), jnp.int32),
            pltpu.VMEM((LISTCAP,), jnp.int32),
            pltpu.VMEM((48,), jnp.int32),
            pltpu.VMEM((16,), jnp.int32),
            pltpu.VMEM((SHARD,), jnp.int32),
            pltpu.VMEM((CC, D), jnp.float32),
            pltpu.VMEM((CC, D), jnp.float32),
            pltpu.VMEM((CC, D), jnp.float32),
            pltpu.VMEM((FNROWS, D), jnp.float32),
            pltpu.VMEM((FNROWS, D), jnp.float32),
            pltpu.SemaphoreType.DMA,
            pltpu.SemaphoreType.DMA,
            pltpu.SemaphoreType.DMA,
            pltpu.SemaphoreType.DMA,
            pltpu.SemaphoreType.DMA,
            pltpu.SemaphoreType.DMA,
            pltpu.SemaphoreType.DMA,
            pltpu.SemaphoreType.DMA,
        ],
    )
    def k(bank_hbm, lab_hbm, fn_hbm, posl_hbm, lidxl_hbm, labl_hbm, lob_hbm,
          cnt_hbm, out_fb, out_lb,
          posl_v, lidxl_v, labl_v, lobuf_v, cnt_v, labsh_v,
          cb0, cb1, cb2, fb0, fb1,
          rs0, rs1, rs2, ws0, ws1, ws2, fs0, fs1):
        wid = lax.axis_index("s") * NC + lax.axis_index("c")
        base = wid * SHARD
        is_last = wid == NW - 1
        steps = jnp.where(is_last, LAST // CC, MAXSTEPS)
        cbufs, rsems, wsems = (cb0, cb1, cb2), (rs0, rs1, rs2), (ws0, ws1, ws2)
        fnbufs, fnsems = (fb0, fb1), (fs0, fs1)
        iota = _iota16()

        def rd_desc(i, b):
            return pltpu.make_async_copy(
                bank_hbm.at[pl.ds(base + i * CC, CC)], cbufs[b], rsems[b])

        def wr_desc(i, b):
            return pltpu.make_async_copy(
                cbufs[b], out_fb.at[pl.ds(base + i * CC, CC)], wsems[b])

        for i in range(2):
            @pl.when(jnp.int32(i) < steps)
            def _(i=i):
                rd_desc(i, i).start()

        pltpu.sync_copy(posl_hbm.at[wid], posl_v)
        pltpu.sync_copy(lidxl_hbm.at[wid], lidxl_v)
        pltpu.sync_copy(labl_hbm.at[wid], labl_v)
        pltpu.sync_copy(lob_hbm.at[wid], lobuf_v)
        pltpu.sync_copy(cnt_hbm.at[wid], cnt_v)

        @pl.when(jnp.logical_not(is_last))
        def _():
            pltpu.sync_copy(lab_hbm.at[pl.ds(base, SHARD)], labsh_v)

        @pl.when(is_last)
        def _():
            pltpu.sync_copy(lab_hbm.at[pl.ds(base, LAST)],
                            labsh_v.at[pl.ds(0, LAST)])

        nw_cnt = cnt_v[pl.ds(0, 16)][0]

        def region_lo(j):
            vb = (j // 16) * 16
            return lobuf_v[pl.ds(vb, 16)][j % 16]

        def region_lo8(j):
            return pl.multiple_of((region_lo(j) // 8) * 8, 8)

        def fn_n(j):
            return (region_lo(j + 1) - region_lo8(j) + FGR - 1) // FGR

        def fn_gather_start(j, f):
            lo8 = region_lo8(j)

            def sbody(g, _):
                off = pl.multiple_of(lo8 + g * FGR, 8)
                pltpu.make_async_copy(
                    fn_hbm.at[posl_v.at[pl.ds(off, FGR)]],
                    fnbufs[f].at[pl.ds(g * FGR, FGR)], fnsems[f]).start()
                return 0
            lax.fori_loop(0, fn_n(j), sbody, 0)

        def fn_gather_wait(j, f):
            lo8 = region_lo8(j)

            def wbody(g, _):
                pltpu.make_async_copy(
                    fn_hbm.at[posl_v.at[pl.ds(lo8, FGR)]],
                    fnbufs[f].at[pl.ds(0, FGR)], fnsems[f]).wait()
                return 0
            lax.fori_loop(0, fn_n(j), wbody, 0)

        with jax.named_scope("ph_labels"):
            def lab_body(j, _):
                m = (iota + j * 16) < nw_cnt
                lidx = jnp.where(m, lidxl_v[pl.ds(j * 16, 16)], 0)
                lv = labl_v[pl.ds(j * 16, 16)]
                plsc.store_scatter(labsh_v, [lidx], lv, mask=m)
                return 0
            lax.fori_loop(0, (nw_cnt + 15) // 16, lab_body, 0)

        fn_gather_start(0, 0)

        def modify_region(i, b, f):
            lo = region_lo(i)
            lo8 = region_lo8(i)
            hie = region_lo(i + 1)
            cbuf = cbufs[b]
            fnbuf = fnbufs[f]

            iot = _iota16()

            def wbody(r, _):
                rsp = jnp.zeros((16,), jnp.int32) + r
                rowsel = plsc.load_gather(lidxl_v, [rsp]) - i * CC
                frsel = jnp.zeros((16,), jnp.int32) + (r - lo8)
                acc = jnp.zeros((16,), jnp.float32)
                nws = []
                for kk in range(D // 16):
                    cols = iot + kk * 16
                    o = plsc.load_gather(cbuf, [rowsel, cols])
                    fn = plsc.load_gather(fnbuf, [frsel, cols])
                    nwk = MOM * o + (1.0 - MOM) * fn
                    nws.append(nwk)
                    acc = acc + nwk * nwk
                s = jnp.sum(acc)
                sv = jnp.zeros((16,), jnp.float32) + s
                sq = sv * _rsqrt(sv)
                inv = 1.0 / (sq + 1e-10)
                for kk in range(D // 16):
                    cols = iot + kk * 16
                    plsc.store_scatter(cbuf, [rowsel, cols], nws[kk] * inv)
                return 0
            lax.fori_loop(lo, hie, wbody, 0)

        with jax.named_scope("ph_ring"):
            for i in range(MAXSTEPS):
                b = i % NBUF
                f = i % 2

                @pl.when(jnp.int32(i) < steps)
                def _(i=i, b=b, f=f):
                    rd_desc(i, b).wait()
                    fn_gather_wait(i, f)
                    if i + 1 < MAXSTEPS:
                        @pl.when(jnp.int32(i + 1) < steps)
                        def _():
                            fn_gather_start(i + 1, 1 - f)
                    modify_region(i, b, f)
                    wr_desc(i, b).start()

                if i + 2 < MAXSTEPS:
                    @pl.when(jnp.int32(i + 2) < steps)
                    def _(i=i):
                        if i >= 1:
                            wr_desc(i - 1, (i - 1) % NBUF).wait()
                        rd_desc(i + 2, (i + 2) % NBUF).start()

            for b in range(NBUF):
                @pl.when(jnp.int32(b) < steps)
                def _(b=b):
                    wr_desc(0, b).wait()

        @pl.when(jnp.logical_not(is_last))
        def _():
            pltpu.sync_copy(labsh_v, out_lb.at[pl.ds(base, SHARD)])

        @pl.when(is_last)
        def _():
            pltpu.sync_copy(labsh_v.at[pl.ds(0, LAST)],
                            out_lb.at[pl.ds(base, LAST)])

    return k(bank, labels, fnorm, posl, lidxl, labl, lob, cnts)


def kernel(feature_bank, label_bank, ind, feature, label):
    ind = ind.astype(jnp.int32)
    label = label.astype(jnp.int32)
    posl, lidxl, labl, lob, cnts = _sc_dedup(ind, label)
    fnorm = _tc_norm(feature)
    return _sc_update(feature_bank, label_bank, fnorm,
                      posl, lidxl, labl, lob, cnts)


# speedup vs baseline: 1.2835x; 1.0224x over previous
import functools

import jax
import jax.numpy as jnp
from jax import lax
from jax.experimental import pallas as pl
from jax.experimental.pallas import tpu as pltpu
from jax.experimental.pallas import tpu_sc as plsc

LENGTH = 100000
D = 128
B = 16384
MOM = 0.5

NC, NS, LANES = 2, 16, 16
NW = NC * NS
SHARD = 3200
LAST = LENGTH - (NW - 1) * SHARD
CC = 160
NBUF = 3
MAXSTEPS = SHARD // CC
TPR = CC // 16
FGR = 32
FNROWS = CC + FGR + 8
LISTCAP = SHARD + 256


def _mesh():
    return plsc.VectorSubcoreMesh(core_axis_name="c", subcore_axis_name="s")


def _iota16():
    return lax.iota(jnp.int32, 16)


def _rsqrt(x):
    xi = plsc.bitcast(x, jnp.int32)
    yi = jnp.int32(0x5F3759DF) - lax.shift_right_arithmetic(xi, 1)
    y = plsc.bitcast(yi, jnp.float32)
    for _ in range(3):
        y = y * (1.5 - 0.5 * x * y * y)
    return y


def _sc_dedup(ind, newlab):
    @functools.partial(
        pl.kernel,
        out_type=(
            jax.ShapeDtypeStruct((NW, LISTCAP), jnp.int32),
            jax.ShapeDtypeStruct((NW, LISTCAP), jnp.int32),
            jax.ShapeDtypeStruct((NW, LISTCAP), jnp.int32),
            jax.ShapeDtypeStruct((NW, 48), jnp.int32),
            jax.ShapeDtypeStruct((NW, 16), jnp.int32),
        ),
        mesh=_mesh(),
        compiler_params=pltpu.CompilerParams(needs_layout_passes=False),
        scratch_types=[
            pltpu.VMEM((B,), jnp.int32),
            pltpu.VMEM((B,), jnp.int32),
            pltpu.VMEM((SHARD,), jnp.int32),
            pltpu.VMEM((LISTCAP,), jnp.int32),
            pltpu.VMEM((LISTCAP,), jnp.int32),
            pltpu.VMEM((LISTCAP,), jnp.int32),
            pltpu.VMEM((48,), jnp.int32),
            pltpu.VMEM((16,), jnp.int32),
            pltpu.SemaphoreType.DMA,
            pltpu.SemaphoreType.DMA,
        ],
    )
    def k(ind_hbm, nlab_hbm,
          posl_hbm, lidxl_hbm, labl_hbm, lob_hbm, cnt_hbm,
          ind_v, labf_v, win_v, posl_v, lidxl_v, labl_v, lobuf_v, cnt_v,
          sem_a, sem_b):
        wid = lax.axis_index("s") * NC + lax.axis_index("c")
        base = wid * SHARD
        hi = jnp.minimum(base + SHARD, LENGTH)
        iota = _iota16()
        lane0 = iota == 0

        d1 = pltpu.make_async_copy(ind_hbm, ind_v, sem_a)
        d2 = pltpu.make_async_copy(nlab_hbm, labf_v, sem_b)
        d1.start()
        d2.start()
        d1.wait()

        def zero_body(t, _):
            win_v[pl.ds(t * 16, 16)] = jnp.zeros((16,), jnp.int32)
            return 0
        lax.fori_loop(0, SHARD // 16, zero_body, 0)

        with jax.named_scope("ph_scan"):
            def scan_body(j, _):
                v = ind_v[pl.ds(j * 16, 16)]
                pos1 = iota + (j * 16 + 1)
                m = jnp.logical_and(v >= base, v < hi)
                lidx = jnp.where(m, v - base, 0)
                plsc.store_scatter(win_v, [lidx], pos1, mask=m)
                return 0
            lax.fori_loop(0, B // 16, scan_body, 0)

        d2.wait()
        with jax.named_scope("ph_compact"):
            def cmp_body(t, off):
                @pl.when(t % TPR == 0)
                def _():
                    plsc.store_scatter(
                        lobuf_v, [jnp.zeros((16,), jnp.int32) + t // TPR],
                        jnp.zeros((16,), jnp.int32) + off, mask=lane0)
                wv = win_v[pl.ds(t * 16, 16)]
                m = wv > 0
                mi = m.astype(jnp.int32)
                pos = wv - 1
                tgt = off + plsc.cumsum(mi) - mi
                tgt = jnp.where(m, tgt, 0)
                plsc.store_scatter(posl_v, [tgt], pos, mask=m)
                lrow = iota + t * 16
                plsc.store_scatter(lidxl_v, [tgt], lrow, mask=m)
                lv = plsc.load_gather(labf_v, [jnp.where(m, pos, 0)], mask=m)
                plsc.store_scatter(labl_v, [tgt], lv, mask=m)
                return off + jnp.sum(mi)
            nw_cnt = lax.fori_loop(0, SHARD // 16, cmp_body, jnp.int32(0))

        l0v = lobuf_v[pl.ds(0, 16)]
        lobuf_v[pl.ds(0, 16)] = jnp.where(iota >= MAXSTEPS, nw_cnt, l0v)
        l1v = lobuf_v[pl.ds(16, 16)]
        lobuf_v[pl.ds(16, 16)] = jnp.where(iota + 16 >= MAXSTEPS, nw_cnt, l1v)
        lobuf_v[pl.ds(32, 16)] = jnp.zeros((16,), jnp.int32) + nw_cnt

        p0 = posl_v[pl.ds(0, 16)][0]
        q0 = lidxl_v[pl.ds(0, 16)][0]

        def pad_body(t, _):
            gi = iota + t * 16
            sel = gi >= nw_cnt
            posl_v[pl.ds(t * 16, 16)] = jnp.where(
                sel, p0, posl_v[pl.ds(t * 16, 16)])
            lidxl_v[pl.ds(t * 16, 16)] = jnp.where(
                sel, q0, lidxl_v[pl.ds(t * 16, 16)])
            return 0
        lax.fori_loop(0, LISTCAP // 16, pad_body, 0)

        cnt_v[pl.ds(0, 16)] = jnp.zeros((16,), jnp.int32) + nw_cnt

        outs = [
            pltpu.make_async_copy(posl_v, posl_hbm.at[wid], sem_a),
            pltpu.make_async_copy(lidxl_v, lidxl_hbm.at[wid], sem_a),
            pltpu.make_async_copy(labl_v, labl_hbm.at[wid], sem_a),
            pltpu.make_async_copy(lobuf_v, lob_hbm.at[wid], sem_a),
            pltpu.make_async_copy(cnt_v, cnt_hbm.at[wid], sem_a),
        ]
        for o in outs:
            o.start()
        for o in outs:
            o.wait()

    return k(ind, newlab)


def _tc_norm(feature):
    RB = 2048

    def body(f_ref, out_ref):
        f = f_ref[...]
        out_ref[...] = f / (jnp.sqrt(jnp.sum(f * f, axis=1, keepdims=True)) + 1e-10)

    return pl.pallas_call(
        body,
        grid=(B // RB,),
        in_specs=[pl.BlockSpec((RB, D), lambda i: (i, 0))],
        out_specs=pl.BlockSpec((RB, D), lambda i: (i, 0)),
        out_shape=jax.ShapeDtypeStruct((B, D), jnp.float32),
    )(feature)


def _sc_update(bank, labels, fnorm, posl, lidxl, labl, lob, cnts):
    @functools.partial(
        pl.kernel,
        out_type=(
            jax.ShapeDtypeStruct((LENGTH, D), jnp.float32),
            jax.ShapeDtypeStruct((LENGTH,), jnp.int32),
        ),
        mesh=_mesh(),
        compiler_params=pltpu.CompilerParams(needs_layout_passes=False),
        scratch_types=[
            pltpu.VMEM((LISTCAP,), jnp.int32),
            pltpu.VMEM((LISTCAP,), jnp.int32),
            pltpu.VMEM((LISTCAP,), jnp.int32),
            pltpu.VMEM((48,), jnp.int32),
            pltpu.VMEM((16,), jnp.int32),
            pltpu.VMEM((SHARD,), jnp.int32),
            pltpu.VMEM((CC, D), jnp.float32),
            pltpu.VMEM((CC, D), jnp.float32),
            pltpu.VMEM((CC, D), jnp.float32),
            pltpu.VMEM((FNROWS, D), jnp.float32),
            pltpu.VMEM((FNROWS, D), jnp.float32),
            pltpu.SemaphoreType.DMA,
            pltpu.SemaphoreType.DMA,
            pltpu.SemaphoreType.DMA,
            pltpu.SemaphoreType.DMA,
            pltpu.SemaphoreType.DMA,
            pltpu.SemaphoreType.DMA,
            pltpu.SemaphoreType.DMA,
            pltpu.SemaphoreType.DMA,
        ],
    )
    def k(bank_hbm, lab_hbm, fn_hbm, posl_hbm, lidxl_hbm, labl_hbm, lob_hbm,
          cnt_hbm, out_fb, out_lb,
          posl_v, lidxl_v, labl_v, lobuf_v, cnt_v, labsh_v,
          cb0, cb1, cb2, fb0, fb1,
          rs0, rs1, rs2, ws0, ws1, ws2, fs0, fs1):
        wid = lax.axis_index("s") * NC + lax.axis_index("c")
        base = wid * SHARD
        is_last = wid == NW - 1
        steps = jnp.where(is_last, LAST // CC, MAXSTEPS)
        cbufs, rsems, wsems = (cb0, cb1, cb2), (rs0, rs1, rs2), (ws0, ws1, ws2)
        fnbufs, fnsems = (fb0, fb1), (fs0, fs1)
        iota = _iota16()

        def rd_desc(i, b):
            return pltpu.make_async_copy(
                bank_hbm.at[pl.ds(base + i * CC, CC)], cbufs[b], rsems[b])

        def wr_desc(i, b):
            return pltpu.make_async_copy(
                cbufs[b], out_fb.at[pl.ds(base + i * CC, CC)], wsems[b])

        for i in range(2):
            @pl.when(jnp.int32(i) < steps)
            def _(i=i):
                rd_desc(i, i).start()

        ins = [
            pltpu.make_async_copy(posl_hbm.at[wid], posl_v, fs0),
            pltpu.make_async_copy(lidxl_hbm.at[wid], lidxl_v, fs0),
            pltpu.make_async_copy(labl_hbm.at[wid], labl_v, fs0),
            pltpu.make_async_copy(lob_hbm.at[wid], lobuf_v, fs0),
            pltpu.make_async_copy(cnt_hbm.at[wid], cnt_v, fs0),
        ]
        for o in ins:
            o.start()

        @pl.when(jnp.logical_not(is_last))
        def _():
            pltpu.make_async_copy(
                lab_hbm.at[pl.ds(base, SHARD)], labsh_v, fs1).start()

        @pl.when(is_last)
        def _():
            pltpu.make_async_copy(
                lab_hbm.at[pl.ds(base, LAST)],
                labsh_v.at[pl.ds(0, LAST)], fs1).start()

        for o in ins:
            o.wait()

        @pl.when(jnp.logical_not(is_last))
        def _():
            pltpu.make_async_copy(
                lab_hbm.at[pl.ds(base, SHARD)], labsh_v, fs1).wait()

        @pl.when(is_last)
        def _():
            pltpu.make_async_copy(
                lab_hbm.at[pl.ds(base, LAST)],
                labsh_v.at[pl.ds(0, LAST)], fs1).wait()

        nw_cnt = cnt_v[pl.ds(0, 16)][0]

        def region_lo(j):
            vb = (j // 16) * 16
            return lobuf_v[pl.ds(vb, 16)][j % 16]

        def region_lo8(j):
            return pl.multiple_of((region_lo(j) // 8) * 8, 8)

        def fn_n(j):
            return (region_lo(j + 1) - region_lo8(j) + FGR - 1) // FGR

        def fn_gather_start(j, f):
            lo8 = region_lo8(j)

            def sbody(g, _):
                off = pl.multiple_of(lo8 + g * FGR, 8)
                pltpu.make_async_copy(
                    fn_hbm.at[posl_v.at[pl.ds(off, FGR)]],
                    fnbufs[f].at[pl.ds(g * FGR, FGR)], fnsems[f]).start()
                return 0
            lax.fori_loop(0, fn_n(j), sbody, 0)

        def fn_gather_wait(j, f):
            lo8 = region_lo8(j)

            def wbody(g, _):
                pltpu.make_async_copy(
                    fn_hbm.at[posl_v.at[pl.ds(lo8, FGR)]],
                    fnbufs[f].at[pl.ds(0, FGR)], fnsems[f]).wait()
                return 0
            lax.fori_loop(0, fn_n(j), wbody, 0)

        with jax.named_scope("ph_labels"):
            def lab_body(j, _):
                m = (iota + j * 16) < nw_cnt
                lidx = jnp.where(m, lidxl_v[pl.ds(j * 16, 16)], 0)
                lv = labl_v[pl.ds(j * 16, 16)]
                plsc.store_scatter(labsh_v, [lidx], lv, mask=m)
                return 0
            lax.fori_loop(0, (nw_cnt + 15) // 16, lab_body, 0)

        fn_gather_start(0, 0)

        def modify_region(i, b, f):
            lo = region_lo(i)
            lo8 = region_lo8(i)
            hie = region_lo(i + 1)
            cbuf = cbufs[b]
            fnbuf = fnbufs[f]

            iot = _iota16()

            def wbody(r, _):
                rsp = jnp.zeros((16,), jnp.int32) + r
                rowsel = plsc.load_gather(lidxl_v, [rsp]) - i * CC
                frsel = jnp.zeros((16,), jnp.int32) + (r - lo8)
                acc = jnp.zeros((16,), jnp.float32)
                nws = []
                for kk in range(D // 16):
                    cols = iot + kk * 16
                    o = plsc.load_gather(cbuf, [rowsel, cols])
                    fn = plsc.load_gather(fnbuf, [frsel, cols])
                    nwk = MOM * o + (1.0 - MOM) * fn
                    nws.append(nwk)
                    acc = acc + nwk * nwk
                s = jnp.sum(acc)
                sv = jnp.zeros((16,), jnp.float32) + s
                sq = sv * _rsqrt(sv)
                inv = 1.0 / (sq + 1e-10)
                for kk in range(D // 16):
                    cols = iot + kk * 16
                    plsc.store_scatter(cbuf, [rowsel, cols], nws[kk] * inv)
                return 0
            lax.fori_loop(lo, hie, wbody, 0)

        with jax.named_scope("ph_ring"):
            for i in range(MAXSTEPS):
                b = i % NBUF
                f = i % 2

                @pl.when(jnp.int32(i) < steps)
                def _(i=i, b=b, f=f):
                    rd_desc(i, b).wait()
                    fn_gather_wait(i, f)
                    if i + 1 < MAXSTEPS:
                        @pl.when(jnp.int32(i + 1) < steps)
                        def _():
                            fn_gather_start(i + 1, 1 - f)
                    modify_region(i, b, f)
                    wr_desc(i, b).start()

                if i + 2 < MAXSTEPS:
                    @pl.when(jnp.int32(i + 2) < steps)
                    def _(i=i):
                        if i >= 1:
                            wr_desc(i - 1, (i - 1) % NBUF).wait()
                        rd_desc(i + 2, (i + 2) % NBUF).start()

            for b in range(NBUF):
                @pl.when(jnp.int32(b) < steps)
                def _(b=b):
                    wr_desc(0, b).wait()

        @pl.when(jnp.logical_not(is_last))
        def _():
            pltpu.sync_copy(labsh_v, out_lb.at[pl.ds(base, SHARD)])

        @pl.when(is_last)
        def _():
            pltpu.sync_copy(labsh_v.at[pl.ds(0, LAST)],
                            out_lb.at[pl.ds(base, LAST)])

    return k(bank, labels, fnorm, posl, lidxl, labl, lob, cnts)


def kernel(feature_bank, label_bank, ind, feature, label):
    ind = ind.astype(jnp.int32)
    label = label.astype(jnp.int32)
    posl, lidxl, labl, lob, cnts = _sc_dedup(ind, label)
    fnorm = _tc_norm(feature)
    return _sc_update(feature_bank, label_bank, fnorm,
                      posl, lidxl, labl, lob, cnts)


# speedup vs baseline: 1.3158x; 1.0251x over previous
import functools

import jax
import jax.numpy as jnp
from jax import lax
from jax.experimental import pallas as pl
from jax.experimental.pallas import tpu as pltpu
from jax.experimental.pallas import tpu_sc as plsc

LENGTH = 100000
D = 128
B = 16384
MOM = 0.5

NC, NS, LANES = 2, 16, 16
NW = NC * NS
SHARD = 3200
LAST = LENGTH - (NW - 1) * SHARD
CC = 160
NBUF = 3
MAXSTEPS = SHARD // CC
TPR = CC // 16
FGR = 32
FNROWS = CC + FGR + 8
LISTCAP = SHARD + 256


def _mesh():
    return plsc.VectorSubcoreMesh(core_axis_name="c", subcore_axis_name="s")


def _iota16():
    return lax.iota(jnp.int32, 16)


def _rsqrt(x):
    xi = plsc.bitcast(x, jnp.int32)
    yi = jnp.int32(0x5F3759DF) - lax.shift_right_arithmetic(xi, 1)
    y = plsc.bitcast(yi, jnp.float32)
    for _ in range(3):
        y = y * (1.5 - 0.5 * x * y * y)
    return y


def _sc_dedup(ind, newlab):
    @functools.partial(
        pl.kernel,
        out_type=(
            jax.ShapeDtypeStruct((NW, LISTCAP), jnp.int32),
            jax.ShapeDtypeStruct((NW, LISTCAP), jnp.int32),
            jax.ShapeDtypeStruct((NW, LISTCAP), jnp.int32),
            jax.ShapeDtypeStruct((NW, 48), jnp.int32),
            jax.ShapeDtypeStruct((NW, 16), jnp.int32),
        ),
        mesh=_mesh(),
        compiler_params=pltpu.CompilerParams(needs_layout_passes=False),
        scratch_types=[
            pltpu.VMEM((B,), jnp.int32),
            pltpu.VMEM((B,), jnp.int32),
            pltpu.VMEM((SHARD,), jnp.int32),
            pltpu.VMEM((LISTCAP,), jnp.int32),
            pltpu.VMEM((LISTCAP,), jnp.int32),
            pltpu.VMEM((LISTCAP,), jnp.int32),
            pltpu.VMEM((48,), jnp.int32),
            pltpu.VMEM((16,), jnp.int32),
            pltpu.SemaphoreType.DMA,
            pltpu.SemaphoreType.DMA,
        ],
    )
    def k(ind_hbm, nlab_hbm,
          posl_hbm, lidxl_hbm, labl_hbm, lob_hbm, cnt_hbm,
          ind_v, labf_v, win_v, posl_v, lidxl_v, labl_v, lobuf_v, cnt_v,
          sem_a, sem_b):
        wid = lax.axis_index("s") * NC + lax.axis_index("c")
        base = wid * SHARD
        hi = jnp.minimum(base + SHARD, LENGTH)
        iota = _iota16()
        lane0 = iota == 0

        d1 = pltpu.make_async_copy(ind_hbm, ind_v, sem_a)
        d2 = pltpu.make_async_copy(nlab_hbm, labf_v, sem_b)
        d1.start()
        d2.start()
        d1.wait()

        def zero_body(t, _):
            win_v[pl.ds(t * 16, 16)] = jnp.zeros((16,), jnp.int32)
            return 0
        lax.fori_loop(0, SHARD // 16, zero_body, 0)

        with jax.named_scope("ph_scan"):
            def scan_body(j, _):
                for u in range(2):
                    v = ind_v[pl.ds(j * 32 + u * 16, 16)]
                    pos1 = iota + (j * 32 + u * 16 + 1)
                    m = jnp.logical_and(v >= base, v < hi)
                    lidx = jnp.where(m, v - base, 0)
                    plsc.store_scatter(win_v, [lidx], pos1, mask=m)
                return 0
            lax.fori_loop(0, B // 32, scan_body, 0)

        d2.wait()
        with jax.named_scope("ph_compact"):
            def cmp_body(t, off):
                @pl.when(t % TPR == 0)
                def _():
                    plsc.store_scatter(
                        lobuf_v, [jnp.zeros((16,), jnp.int32) + t // TPR],
                        jnp.zeros((16,), jnp.int32) + off, mask=lane0)
                wv = win_v[pl.ds(t * 16, 16)]
                m = wv > 0
                mi = m.astype(jnp.int32)
                pos = wv - 1
                tgt = off + plsc.cumsum(mi) - mi
                tgt = jnp.where(m, tgt, 0)
                plsc.store_scatter(posl_v, [tgt], pos, mask=m)
                lrow = iota + t * 16
                plsc.store_scatter(lidxl_v, [tgt], lrow, mask=m)
                lv = plsc.load_gather(labf_v, [jnp.where(m, pos, 0)], mask=m)
                plsc.store_scatter(labl_v, [tgt], lv, mask=m)
                return off + jnp.sum(mi)
            nw_cnt = lax.fori_loop(0, SHARD // 16, cmp_body, jnp.int32(0))

        l0v = lobuf_v[pl.ds(0, 16)]
        lobuf_v[pl.ds(0, 16)] = jnp.where(iota >= MAXSTEPS, nw_cnt, l0v)
        l1v = lobuf_v[pl.ds(16, 16)]
        lobuf_v[pl.ds(16, 16)] = jnp.where(iota + 16 >= MAXSTEPS, nw_cnt, l1v)
        lobuf_v[pl.ds(32, 16)] = jnp.zeros((16,), jnp.int32) + nw_cnt

        p0 = posl_v[pl.ds(0, 16)][0]
        q0 = lidxl_v[pl.ds(0, 16)][0]

        def pad_body(t, _):
            gi = iota + t * 16
            sel = gi >= nw_cnt
            posl_v[pl.ds(t * 16, 16)] = jnp.where(
                sel, p0, posl_v[pl.ds(t * 16, 16)])
            lidxl_v[pl.ds(t * 16, 16)] = jnp.where(
                sel, q0, lidxl_v[pl.ds(t * 16, 16)])
            return 0
        lax.fori_loop(0, LISTCAP // 16, pad_body, 0)

        cnt_v[pl.ds(0, 16)] = jnp.zeros((16,), jnp.int32) + nw_cnt

        outs = [
            pltpu.make_async_copy(posl_v, posl_hbm.at[wid], sem_a),
            pltpu.make_async_copy(lidxl_v, lidxl_hbm.at[wid], sem_a),
            pltpu.make_async_copy(labl_v, labl_hbm.at[wid], sem_a),
            pltpu.make_async_copy(lobuf_v, lob_hbm.at[wid], sem_a),
            pltpu.make_async_copy(cnt_v, cnt_hbm.at[wid], sem_a),
        ]
        for o in outs:
            o.start()
        for o in outs:
            o.wait()

    return k(ind, newlab)


def _tc_norm(feature):
    RB = 2048

    def body(f_ref, out_ref):
        f = f_ref[...]
        out_ref[...] = f / (jnp.sqrt(jnp.sum(f * f, axis=1, keepdims=True)) + 1e-10)

    return pl.pallas_call(
        body,
        grid=(B // RB,),
        in_specs=[pl.BlockSpec((RB, D), lambda i: (i, 0))],
        out_specs=pl.BlockSpec((RB, D), lambda i: (i, 0)),
        out_shape=jax.ShapeDtypeStruct((B, D), jnp.float32),
    )(feature)


def _sc_update(bank, labels, fnorm, posl, lidxl, labl, lob, cnts):
    @functools.partial(
        pl.kernel,
        out_type=(
            jax.ShapeDtypeStruct((LENGTH, D), jnp.float32),
            jax.ShapeDtypeStruct((LENGTH,), jnp.int32),
        ),
        mesh=_mesh(),
        compiler_params=pltpu.CompilerParams(needs_layout_passes=False),
        scratch_types=[
            pltpu.VMEM((LISTCAP,), jnp.int32),
            pltpu.VMEM((LISTCAP,), jnp.int32),
            pltpu.VMEM((LISTCAP,), jnp.int32),
            pltpu.VMEM((48,), jnp.int32),
            pltpu.VMEM((16,), jnp.int32),
            pltpu.VMEM((SHARD,), jnp.int32),
            pltpu.VMEM((CC, D), jnp.float32),
            pltpu.VMEM((CC, D), jnp.float32),
            pltpu.VMEM((CC, D), jnp.float32),
            pltpu.VMEM((FNROWS, D), jnp.float32),
            pltpu.VMEM((FNROWS, D), jnp.float32),
            pltpu.SemaphoreType.DMA,
            pltpu.SemaphoreType.DMA,
            pltpu.SemaphoreType.DMA,
            pltpu.SemaphoreType.DMA,
            pltpu.SemaphoreType.DMA,
            pltpu.SemaphoreType.DMA,
            pltpu.SemaphoreType.DMA,
            pltpu.SemaphoreType.DMA,
            pltpu.SemaphoreType.DMA,
        ],
    )
    def k(bank_hbm, lab_hbm, fn_hbm, posl_hbm, lidxl_hbm, labl_hbm, lob_hbm,
          cnt_hbm, out_fb, out_lb,
          posl_v, lidxl_v, labl_v, lobuf_v, cnt_v, labsh_v,
          cb0, cb1, cb2, fb0, fb1,
          rs0, rs1, rs2, ws0, ws1, ws2, fs0, fs1, lsem):
        wid = lax.axis_index("s") * NC + lax.axis_index("c")
        base = wid * SHARD
        is_last = wid == NW - 1
        steps = jnp.where(is_last, LAST // CC, MAXSTEPS)
        cbufs, rsems, wsems = (cb0, cb1, cb2), (rs0, rs1, rs2), (ws0, ws1, ws2)
        fnbufs, fnsems = (fb0, fb1), (fs0, fs1)
        iota = _iota16()

        def rd_desc(i, b):
            return pltpu.make_async_copy(
                bank_hbm.at[pl.ds(base + i * CC, CC)], cbufs[b], rsems[b])

        def wr_desc(i, b):
            return pltpu.make_async_copy(
                cbufs[b], out_fb.at[pl.ds(base + i * CC, CC)], wsems[b])

        for i in range(2):
            @pl.when(jnp.int32(i) < steps)
            def _(i=i):
                rd_desc(i, i).start()

        ins = [
            pltpu.make_async_copy(posl_hbm.at[wid], posl_v, fs0),
            pltpu.make_async_copy(lidxl_hbm.at[wid], lidxl_v, fs0),
            pltpu.make_async_copy(labl_hbm.at[wid], labl_v, fs0),
            pltpu.make_async_copy(lob_hbm.at[wid], lobuf_v, fs0),
            pltpu.make_async_copy(cnt_hbm.at[wid], cnt_v, fs0),
        ]
        for o in ins:
            o.start()

        @pl.when(jnp.logical_not(is_last))
        def _():
            pltpu.make_async_copy(
                lab_hbm.at[pl.ds(base, SHARD)], labsh_v, fs1).start()

        @pl.when(is_last)
        def _():
            pltpu.make_async_copy(
                lab_hbm.at[pl.ds(base, LAST)],
                labsh_v.at[pl.ds(0, LAST)], fs1).start()

        for o in ins:
            o.wait()

        @pl.when(jnp.logical_not(is_last))
        def _():
            pltpu.make_async_copy(
                lab_hbm.at[pl.ds(base, SHARD)], labsh_v, fs1).wait()

        @pl.when(is_last)
        def _():
            pltpu.make_async_copy(
                lab_hbm.at[pl.ds(base, LAST)],
                labsh_v.at[pl.ds(0, LAST)], fs1).wait()

        nw_cnt = cnt_v[pl.ds(0, 16)][0]

        def region_lo(j):
            vb = (j // 16) * 16
            return lobuf_v[pl.ds(vb, 16)][j % 16]

        def region_lo8(j):
            return pl.multiple_of((region_lo(j) // 8) * 8, 8)

        def fn_n(j):
            return (region_lo(j + 1) - region_lo8(j) + FGR - 1) // FGR

        def fn_gather_start(j, f):
            lo8 = region_lo8(j)

            def sbody(g, _):
                off = pl.multiple_of(lo8 + g * FGR, 8)
                pltpu.make_async_copy(
                    fn_hbm.at[posl_v.at[pl.ds(off, FGR)]],
                    fnbufs[f].at[pl.ds(g * FGR, FGR)], fnsems[f]).start()
                return 0
            lax.fori_loop(0, fn_n(j), sbody, 0)

        def fn_gather_wait(j, f):
            lo8 = region_lo8(j)

            def wbody(g, _):
                pltpu.make_async_copy(
                    fn_hbm.at[posl_v.at[pl.ds(lo8, FGR)]],
                    fnbufs[f].at[pl.ds(0, FGR)], fnsems[f]).wait()
                return 0
            lax.fori_loop(0, fn_n(j), wbody, 0)

        with jax.named_scope("ph_labels"):
            def lab_body(j, _):
                m = (iota + j * 16) < nw_cnt
                lidx = jnp.where(m, lidxl_v[pl.ds(j * 16, 16)], 0)
                lv = labl_v[pl.ds(j * 16, 16)]
                plsc.store_scatter(labsh_v, [lidx], lv, mask=m)
                return 0
            lax.fori_loop(0, (nw_cnt + 15) // 16, lab_body, 0)

        @pl.when(jnp.logical_not(is_last))
        def _():
            pltpu.make_async_copy(
                labsh_v, out_lb.at[pl.ds(base, SHARD)], lsem).start()

        @pl.when(is_last)
        def _():
            pltpu.make_async_copy(
                labsh_v.at[pl.ds(0, LAST)],
                out_lb.at[pl.ds(base, LAST)], lsem).start()

        fn_gather_start(0, 0)

        def modify_region(i, b, f):
            lo = region_lo(i)
            lo8 = region_lo8(i)
            hie = region_lo(i + 1)
            cbuf = cbufs[b]
            fnbuf = fnbufs[f]

            iot = _iota16()

            def wbody(r, _):
                rsp = jnp.zeros((16,), jnp.int32) + r
                rowsel = plsc.load_gather(lidxl_v, [rsp]) - i * CC
                frsel = jnp.zeros((16,), jnp.int32) + (r - lo8)
                acc = jnp.zeros((16,), jnp.float32)
                nws = []
                for kk in range(D // 16):
                    cols = iot + kk * 16
                    o = plsc.load_gather(cbuf, [rowsel, cols])
                    fn = plsc.load_gather(fnbuf, [frsel, cols])
                    nwk = MOM * o + (1.0 - MOM) * fn
                    nws.append(nwk)
                    acc = acc + nwk * nwk
                s = jnp.sum(acc)
                sv = jnp.zeros((16,), jnp.float32) + s
                sq = sv * _rsqrt(sv)
                inv = 1.0 / (sq + 1e-10)
                for kk in range(D // 16):
                    cols = iot + kk * 16
                    plsc.store_scatter(cbuf, [rowsel, cols], nws[kk] * inv)
                return 0
            lax.fori_loop(lo, hie, wbody, 0)

        with jax.named_scope("ph_ring"):
            for i in range(MAXSTEPS):
                b = i % NBUF
                f = i % 2

                @pl.when(jnp.int32(i) < steps)
                def _(i=i, b=b, f=f):
                    rd_desc(i, b).wait()
                    fn_gather_wait(i, f)
                    if i + 1 < MAXSTEPS:
                        @pl.when(jnp.int32(i + 1) < steps)
                        def _():
                            fn_gather_start(i + 1, 1 - f)
                    modify_region(i, b, f)
                    wr_desc(i, b).start()

                if i + 2 < MAXSTEPS:
                    @pl.when(jnp.int32(i + 2) < steps)
                    def _(i=i):
                        if i >= 1:
                            wr_desc(i - 1, (i - 1) % NBUF).wait()
                        rd_desc(i + 2, (i + 2) % NBUF).start()

            for b in range(NBUF):
                @pl.when(jnp.int32(b) < steps)
                def _(b=b):
                    wr_desc(0, b).wait()

        @pl.when(jnp.logical_not(is_last))
        def _():
            pltpu.make_async_copy(
                labsh_v, out_lb.at[pl.ds(base, SHARD)], lsem).wait()

        @pl.when(is_last)
        def _():
            pltpu.make_async_copy(
                labsh_v.at[pl.ds(0, LAST)],
                out_lb.at[pl.ds(base, LAST)], lsem).wait()

    return k(bank, labels, fnorm, posl, lidxl, labl, lob, cnts)


def kernel(feature_bank, label_bank, ind, feature, label):
    ind = ind.astype(jnp.int32)
    label = label.astype(jnp.int32)
    posl, lidxl, labl, lob, cnts = _sc_dedup(ind, label)
    fnorm = _tc_norm(feature)
    return _sc_update(feature_bank, label_bank, fnorm,
                      posl, lidxl, labl, lob, cnts)


# speedup vs baseline: 1.3316x; 1.0120x over previous
import functools

import jax
import jax.numpy as jnp
from jax import lax
from jax.experimental import pallas as pl
from jax.experimental.pallas import tpu as pltpu
from jax.experimental.pallas import tpu_sc as plsc

LENGTH = 100000
D = 128
B = 16384
MOM = 0.5

NC, NS, LANES = 2, 16, 16
NW = NC * NS
SHARD = 3200
LAST = LENGTH - (NW - 1) * SHARD
CC = 160
NBUF = 3
MAXSTEPS = SHARD // CC
TPR = CC // 16
FGR = 32
FNROWS = CC + FGR + 8
LISTCAP = SHARD + 256


def _mesh():
    return plsc.VectorSubcoreMesh(core_axis_name="c", subcore_axis_name="s")


def _iota16():
    return lax.iota(jnp.int32, 16)


def _rsqrt(x):
    xi = plsc.bitcast(x, jnp.int32)
    yi = jnp.int32(0x5F3759DF) - lax.shift_right_arithmetic(xi, 1)
    y = plsc.bitcast(yi, jnp.float32)
    for _ in range(3):
        y = y * (1.5 - 0.5 * x * y * y)
    return y


def _sc_dedup(ind, newlab):
    @functools.partial(
        pl.kernel,
        out_type=(
            jax.ShapeDtypeStruct((NW, LISTCAP), jnp.int32),
            jax.ShapeDtypeStruct((NW, LISTCAP), jnp.int32),
            jax.ShapeDtypeStruct((NW, LISTCAP), jnp.int32),
            jax.ShapeDtypeStruct((NW, 48), jnp.int32),
            jax.ShapeDtypeStruct((NW, 16), jnp.int32),
        ),
        mesh=_mesh(),
        compiler_params=pltpu.CompilerParams(needs_layout_passes=False),
        scratch_types=[
            pltpu.VMEM((B,), jnp.int32),
            pltpu.VMEM((B,), jnp.int32),
            pltpu.VMEM((SHARD,), jnp.int32),
            pltpu.VMEM((LISTCAP,), jnp.int32),
            pltpu.VMEM((LISTCAP,), jnp.int32),
            pltpu.VMEM((LISTCAP,), jnp.int32),
            pltpu.VMEM((48,), jnp.int32),
            pltpu.VMEM((16,), jnp.int32),
            pltpu.SemaphoreType.DMA,
            pltpu.SemaphoreType.DMA,
        ],
    )
    def k(ind_hbm, nlab_hbm,
          posl_hbm, lidxl_hbm, labl_hbm, lob_hbm, cnt_hbm,
          ind_v, labf_v, win_v, posl_v, lidxl_v, labl_v, lobuf_v, cnt_v,
          sem_a, sem_b):
        wid = lax.axis_index("s") * NC + lax.axis_index("c")
        base = wid * SHARD
        hi = jnp.minimum(base + SHARD, LENGTH)
        iota = _iota16()
        lane0 = iota == 0

        QB = B // 4
        ind_descs = [
            pltpu.make_async_copy(ind_hbm.at[pl.ds(c * QB, QB)],
                                  ind_v.at[pl.ds(c * QB, QB)], sem_a)
            for c in range(4)
        ]
        for dsc in ind_descs:
            dsc.start()
        d2 = pltpu.make_async_copy(nlab_hbm, labf_v, sem_b)
        d2.start()

        def zero_body(t, _):
            win_v[pl.ds(t * 16, 16)] = jnp.zeros((16,), jnp.int32)
            return 0
        lax.fori_loop(0, SHARD // 16, zero_body, 0)

        with jax.named_scope("ph_scan"):
            def scan_body(j, _):
                for u in range(2):
                    v = ind_v[pl.ds(j * 32 + u * 16, 16)]
                    pos1 = iota + (j * 32 + u * 16 + 1)
                    m = jnp.logical_and(v >= base, v < hi)
                    lidx = jnp.where(m, v - base, 0)
                    plsc.store_scatter(win_v, [lidx], pos1, mask=m)
                return 0
            for c in range(4):
                ind_descs[c].wait()
                lax.fori_loop(c * (QB // 32), (c + 1) * (QB // 32),
                              scan_body, 0)

        d2.wait()
        with jax.named_scope("ph_compact"):
            def cmp_body(t, off):
                @pl.when(t % TPR == 0)
                def _():
                    plsc.store_scatter(
                        lobuf_v, [jnp.zeros((16,), jnp.int32) + t // TPR],
                        jnp.zeros((16,), jnp.int32) + off, mask=lane0)
                wv = win_v[pl.ds(t * 16, 16)]
                m = wv > 0
                mi = m.astype(jnp.int32)
                pos = wv - 1
                tgt = off + plsc.cumsum(mi) - mi
                tgt = jnp.where(m, tgt, 0)
                plsc.store_scatter(posl_v, [tgt], pos, mask=m)
                lrow = iota + t * 16
                plsc.store_scatter(lidxl_v, [tgt], lrow, mask=m)
                lv = plsc.load_gather(labf_v, [jnp.where(m, pos, 0)], mask=m)
                plsc.store_scatter(labl_v, [tgt], lv, mask=m)
                return off + jnp.sum(mi)
            nw_cnt = lax.fori_loop(0, SHARD // 16, cmp_body, jnp.int32(0))

        l0v = lobuf_v[pl.ds(0, 16)]
        lobuf_v[pl.ds(0, 16)] = jnp.where(iota >= MAXSTEPS, nw_cnt, l0v)
        l1v = lobuf_v[pl.ds(16, 16)]
        lobuf_v[pl.ds(16, 16)] = jnp.where(iota + 16 >= MAXSTEPS, nw_cnt, l1v)
        lobuf_v[pl.ds(32, 16)] = jnp.zeros((16,), jnp.int32) + nw_cnt

        p0 = posl_v[pl.ds(0, 16)][0]
        q0 = lidxl_v[pl.ds(0, 16)][0]

        def pad_body(t, _):
            gi = iota + t * 16
            sel = gi >= nw_cnt
            posl_v[pl.ds(t * 16, 16)] = jnp.where(
                sel, p0, posl_v[pl.ds(t * 16, 16)])
            lidxl_v[pl.ds(t * 16, 16)] = jnp.where(
                sel, q0, lidxl_v[pl.ds(t * 16, 16)])
            return 0
        lax.fori_loop(0, LISTCAP // 16, pad_body, 0)

        cnt_v[pl.ds(0, 16)] = jnp.zeros((16,), jnp.int32) + nw_cnt

        outs = [
            pltpu.make_async_copy(posl_v, posl_hbm.at[wid], sem_a),
            pltpu.make_async_copy(lidxl_v, lidxl_hbm.at[wid], sem_a),
            pltpu.make_async_copy(labl_v, labl_hbm.at[wid], sem_a),
            pltpu.make_async_copy(lobuf_v, lob_hbm.at[wid], sem_a),
            pltpu.make_async_copy(cnt_v, cnt_hbm.at[wid], sem_a),
        ]
        for o in outs:
            o.start()
        for o in outs:
            o.wait()

    return k(ind, newlab)


def _tc_norm(feature):
    RB = 2048

    def body(f_ref, out_ref):
        f = f_ref[...]
        out_ref[...] = f / (jnp.sqrt(jnp.sum(f * f, axis=1, keepdims=True)) + 1e-10)

    return pl.pallas_call(
        body,
        grid=(B // RB,),
        in_specs=[pl.BlockSpec((RB, D), lambda i: (i, 0))],
        out_specs=pl.BlockSpec((RB, D), lambda i: (i, 0)),
        out_shape=jax.ShapeDtypeStruct((B, D), jnp.float32),
    )(feature)


def _sc_update(bank, labels, fnorm, posl, lidxl, labl, lob, cnts):
    @functools.partial(
        pl.kernel,
        out_type=(
            jax.ShapeDtypeStruct((LENGTH, D), jnp.float32),
            jax.ShapeDtypeStruct((LENGTH,), jnp.int32),
        ),
        mesh=_mesh(),
        compiler_params=pltpu.CompilerParams(needs_layout_passes=False),
        scratch_types=[
            pltpu.VMEM((LISTCAP,), jnp.int32),
            pltpu.VMEM((LISTCAP,), jnp.int32),
            pltpu.VMEM((LISTCAP,), jnp.int32),
            pltpu.VMEM((48,), jnp.int32),
            pltpu.VMEM((16,), jnp.int32),
            pltpu.VMEM((SHARD,), jnp.int32),
            pltpu.VMEM((CC, D), jnp.float32),
            pltpu.VMEM((CC, D), jnp.float32),
            pltpu.VMEM((CC, D), jnp.float32),
            pltpu.VMEM((FNROWS, D), jnp.float32),
            pltpu.VMEM((FNROWS, D), jnp.float32),
            pltpu.SemaphoreType.DMA,
            pltpu.SemaphoreType.DMA,
            pltpu.SemaphoreType.DMA,
            pltpu.SemaphoreType.DMA,
            pltpu.SemaphoreType.DMA,
            pltpu.SemaphoreType.DMA,
            pltpu.SemaphoreType.DMA,
            pltpu.SemaphoreType.DMA,
            pltpu.SemaphoreType.DMA,
        ],
    )
    def k(bank_hbm, lab_hbm, fn_hbm, posl_hbm, lidxl_hbm, labl_hbm, lob_hbm,
          cnt_hbm, out_fb, out_lb,
          posl_v, lidxl_v, labl_v, lobuf_v, cnt_v, labsh_v,
          cb0, cb1, cb2, fb0, fb1,
          rs0, rs1, rs2, ws0, ws1, ws2, fs0, fs1, lsem):
        wid = lax.axis_index("s") * NC + lax.axis_index("c")
        base = wid * SHARD
        is_last = wid == NW - 1
        steps = jnp.where(is_last, LAST // CC, MAXSTEPS)
        cbufs, rsems, wsems = (cb0, cb1, cb2), (rs0, rs1, rs2), (ws0, ws1, ws2)
        fnbufs, fnsems = (fb0, fb1), (fs0, fs1)
        iota = _iota16()

        def rd_desc(i, b):
            return pltpu.make_async_copy(
                bank_hbm.at[pl.ds(base + i * CC, CC)], cbufs[b], rsems[b])

        def wr_desc(i, b):
            return pltpu.make_async_copy(
                cbufs[b], out_fb.at[pl.ds(base + i * CC, CC)], wsems[b])

        for i in range(2):
            @pl.when(jnp.int32(i) < steps)
            def _(i=i):
                rd_desc(i, i).start()

        ins = [
            pltpu.make_async_copy(posl_hbm.at[wid], posl_v, fs0),
            pltpu.make_async_copy(lidxl_hbm.at[wid], lidxl_v, fs0),
            pltpu.make_async_copy(labl_hbm.at[wid], labl_v, fs0),
            pltpu.make_async_copy(lob_hbm.at[wid], lobuf_v, fs0),
            pltpu.make_async_copy(cnt_hbm.at[wid], cnt_v, fs0),
        ]
        for o in ins:
            o.start()

        @pl.when(jnp.logical_not(is_last))
        def _():
            pltpu.make_async_copy(
                lab_hbm.at[pl.ds(base, SHARD)], labsh_v, fs1).start()

        @pl.when(is_last)
        def _():
            pltpu.make_async_copy(
                lab_hbm.at[pl.ds(base, LAST)],
                labsh_v.at[pl.ds(0, LAST)], fs1).start()

        for o in ins:
            o.wait()

        @pl.when(jnp.logical_not(is_last))
        def _():
            pltpu.make_async_copy(
                lab_hbm.at[pl.ds(base, SHARD)], labsh_v, fs1).wait()

        @pl.when(is_last)
        def _():
            pltpu.make_async_copy(
                lab_hbm.at[pl.ds(base, LAST)],
                labsh_v.at[pl.ds(0, LAST)], fs1).wait()

        nw_cnt = cnt_v[pl.ds(0, 16)][0]

        def region_lo(j):
            vb = (j // 16) * 16
            return lobuf_v[pl.ds(vb, 16)][j % 16]

        def region_lo8(j):
            return pl.multiple_of((region_lo(j) // 8) * 8, 8)

        def fn_n(j):
            return (region_lo(j + 1) - region_lo8(j) + FGR - 1) // FGR

        def fn_gather_start(j, f):
            lo8 = region_lo8(j)

            def sbody(g, _):
                off = pl.multiple_of(lo8 + g * FGR, 8)
                pltpu.make_async_copy(
                    fn_hbm.at[posl_v.at[pl.ds(off, FGR)]],
                    fnbufs[f].at[pl.ds(g * FGR, FGR)], fnsems[f]).start()
                return 0
            lax.fori_loop(0, fn_n(j), sbody, 0)

        def fn_gather_wait(j, f):
            lo8 = region_lo8(j)

            def wbody(g, _):
                pltpu.make_async_copy(
                    fn_hbm.at[posl_v.at[pl.ds(lo8, FGR)]],
                    fnbufs[f].at[pl.ds(0, FGR)], fnsems[f]).wait()
                return 0
            lax.fori_loop(0, fn_n(j), wbody, 0)

        with jax.named_scope("ph_labels"):
            def lab_body(j, _):
                m = (iota + j * 16) < nw_cnt
                lidx = jnp.where(m, lidxl_v[pl.ds(j * 16, 16)], 0)
                lv = labl_v[pl.ds(j * 16, 16)]
                plsc.store_scatter(labsh_v, [lidx], lv, mask=m)
                return 0
            lax.fori_loop(0, (nw_cnt + 15) // 16, lab_body, 0)

        @pl.when(jnp.logical_not(is_last))
        def _():
            pltpu.make_async_copy(
                labsh_v, out_lb.at[pl.ds(base, SHARD)], lsem).start()

        @pl.when(is_last)
        def _():
            pltpu.make_async_copy(
                labsh_v.at[pl.ds(0, LAST)],
                out_lb.at[pl.ds(base, LAST)], lsem).start()

        fn_gather_start(0, 0)

        def modify_region(i, b, f):
            lo = region_lo(i)
            lo8 = region_lo8(i)
            hie = region_lo(i + 1)
            cbuf = cbufs[b]
            fnbuf = fnbufs[f]

            iot = _iota16()

            def wbody(r, _):
                rsp = jnp.zeros((16,), jnp.int32) + r
                rowsel = plsc.load_gather(lidxl_v, [rsp]) - i * CC
                frsel = jnp.zeros((16,), jnp.int32) + (r - lo8)
                acc = jnp.zeros((16,), jnp.float32)
                nws = []
                for kk in range(D // 16):
                    cols = iot + kk * 16
                    o = plsc.load_gather(cbuf, [rowsel, cols])
                    fn = plsc.load_gather(fnbuf, [frsel, cols])
                    nwk = MOM * o + (1.0 - MOM) * fn
                    nws.append(nwk)
                    acc = acc + nwk * nwk
                s = jnp.sum(acc)
                sv = jnp.zeros((16,), jnp.float32) + s
                sq = sv * _rsqrt(sv)
                inv = 1.0 / (sq + 1e-10)
                for kk in range(D // 16):
                    cols = iot + kk * 16
                    plsc.store_scatter(cbuf, [rowsel, cols], nws[kk] * inv)
                return 0
            lax.fori_loop(lo, hie, wbody, 0)

        with jax.named_scope("ph_ring"):
            for i in range(MAXSTEPS):
                b = i % NBUF
                f = i % 2

                @pl.when(jnp.int32(i) < steps)
                def _(i=i, b=b, f=f):
                    rd_desc(i, b).wait()
                    fn_gather_wait(i, f)
                    if i + 1 < MAXSTEPS:
                        @pl.when(jnp.int32(i + 1) < steps)
                        def _():
                            fn_gather_start(i + 1, 1 - f)
                    modify_region(i, b, f)
                    wr_desc(i, b).start()

                if i + 2 < MAXSTEPS:
                    @pl.when(jnp.int32(i + 2) < steps)
                    def _(i=i):
                        if i >= 1:
                            wr_desc(i - 1, (i - 1) % NBUF).wait()
                        rd_desc(i + 2, (i + 2) % NBUF).start()

            for b in range(NBUF):
                @pl.when(jnp.int32(b) < steps)
                def _(b=b):
                    wr_desc(0, b).wait()

        @pl.when(jnp.logical_not(is_last))
        def _():
            pltpu.make_async_copy(
                labsh_v, out_lb.at[pl.ds(base, SHARD)], lsem).wait()

        @pl.when(is_last)
        def _():
            pltpu.make_async_copy(
                labsh_v.at[pl.ds(0, LAST)],
                out_lb.at[pl.ds(base, LAST)], lsem).wait()

    return k(bank, labels, fnorm, posl, lidxl, labl, lob, cnts)


def kernel(feature_bank, label_bank, ind, feature, label):
    ind = ind.astype(jnp.int32)
    label = label.astype(jnp.int32)
    posl, lidxl, labl, lob, cnts = _sc_dedup(ind, label)
    fnorm = _tc_norm(feature)
    return _sc_update(feature_bank, label_bank, fnorm,
                      posl, lidxl, labl, lob, cnts)


# speedup vs baseline: 1.4446x; 1.0849x over previous
import functools

import jax
import jax.numpy as jnp
from jax import lax
from jax.experimental import pallas as pl
from jax.experimental.pallas import tpu as pltpu
from jax.experimental.pallas import tpu_sc as plsc

LENGTH = 100000
D = 128
B = 16384
MOM = 0.5

NC, NS, LANES = 2, 16, 16
NW = NC * NS
SHARD = 3200
LAST = LENGTH - (NW - 1) * SHARD
CC = 160
NBUF = 3
MAXSTEPS = SHARD // CC
TPR = CC // 16
FGR = 32
FNROWS = CC + FGR + 8
LISTCAP = SHARD + 256


def _mesh():
    return plsc.VectorSubcoreMesh(core_axis_name="c", subcore_axis_name="s")


def _iota16():
    return lax.iota(jnp.int32, 16)


def _rsqrt(x):
    xi = plsc.bitcast(x, jnp.int32)
    yi = jnp.int32(0x5F3759DF) - lax.shift_right_arithmetic(xi, 1)
    y = plsc.bitcast(yi, jnp.float32)
    for _ in range(3):
        y = y * (1.5 - 0.5 * x * y * y)
    return y


def _sc_dedup(ind, newlab):
    @functools.partial(
        pl.kernel,
        out_type=(
            jax.ShapeDtypeStruct((NW, LISTCAP), jnp.int32),
            jax.ShapeDtypeStruct((NW, LISTCAP), jnp.int32),
            jax.ShapeDtypeStruct((NW, LISTCAP), jnp.int32),
            jax.ShapeDtypeStruct((NW, 48), jnp.int32),
            jax.ShapeDtypeStruct((NW, 16), jnp.int32),
        ),
        mesh=_mesh(),
        compiler_params=pltpu.CompilerParams(needs_layout_passes=False),
        scratch_types=[
            pltpu.VMEM((B,), jnp.int32),
            pltpu.VMEM((B,), jnp.int32),
            pltpu.VMEM((SHARD,), jnp.int32),
            pltpu.VMEM((LISTCAP,), jnp.int32),
            pltpu.VMEM((LISTCAP,), jnp.int32),
            pltpu.VMEM((LISTCAP,), jnp.int32),
            pltpu.VMEM((48,), jnp.int32),
            pltpu.VMEM((16,), jnp.int32),
            pltpu.SemaphoreType.DMA,
            pltpu.SemaphoreType.DMA,
        ],
    )
    def k(ind_hbm, nlab_hbm,
          posl_hbm, lidxl_hbm, labl_hbm, lob_hbm, cnt_hbm,
          ind_v, labf_v, win_v, posl_v, lidxl_v, labl_v, lobuf_v, cnt_v,
          sem_a, sem_b):
        wid = lax.axis_index("s") * NC + lax.axis_index("c")
        base = wid * SHARD
        hi = jnp.minimum(base + SHARD, LENGTH)
        iota = _iota16()
        lane0 = iota == 0

        QB = B // 4
        ind_descs = [
            pltpu.make_async_copy(ind_hbm.at[pl.ds(c * QB, QB)],
                                  ind_v.at[pl.ds(c * QB, QB)], sem_a)
            for c in range(4)
        ]
        for dsc in ind_descs:
            dsc.start()
        d2 = pltpu.make_async_copy(nlab_hbm, labf_v, sem_b)
        d2.start()

        def zero_body(t, _):
            win_v[pl.ds(t * 16, 16)] = jnp.zeros((16,), jnp.int32)
            return 0
        lax.fori_loop(0, SHARD // 16, zero_body, 0)

        with jax.named_scope("ph_scan"):
            def scan_body(j, _):
                for u in range(2):
                    v = ind_v[pl.ds(j * 32 + u * 16, 16)]
                    pos1 = iota + (j * 32 + u * 16 + 1)
                    m = jnp.logical_and(v >= base, v < hi)
                    lidx = jnp.where(m, v - base, 0)
                    plsc.store_scatter(win_v, [lidx], pos1, mask=m)
                return 0
            for c in range(4):
                ind_descs[c].wait()
                lax.fori_loop(c * (QB // 32), (c + 1) * (QB // 32),
                              scan_body, 0)

        d2.wait()
        with jax.named_scope("ph_compact"):
            def cmp_body(t, off):
                @pl.when(t % TPR == 0)
                def _():
                    plsc.store_scatter(
                        lobuf_v, [jnp.zeros((16,), jnp.int32) + t // TPR],
                        jnp.zeros((16,), jnp.int32) + off, mask=lane0)
                wv = win_v[pl.ds(t * 16, 16)]
                m = wv > 0
                mi = m.astype(jnp.int32)
                pos = wv - 1
                tgt = off + plsc.cumsum(mi) - mi
                tgt = jnp.where(m, tgt, 0)
                plsc.store_scatter(posl_v, [tgt], pos, mask=m)
                lrow = iota + t * 16
                plsc.store_scatter(lidxl_v, [tgt], lrow, mask=m)
                lv = plsc.load_gather(labf_v, [jnp.where(m, pos, 0)], mask=m)
                plsc.store_scatter(labl_v, [tgt], lv, mask=m)
                return off + jnp.sum(mi)
            nw_cnt = lax.fori_loop(0, SHARD // 16, cmp_body, jnp.int32(0))

        l0v = lobuf_v[pl.ds(0, 16)]
        lobuf_v[pl.ds(0, 16)] = jnp.where(iota >= MAXSTEPS, nw_cnt, l0v)
        l1v = lobuf_v[pl.ds(16, 16)]
        lobuf_v[pl.ds(16, 16)] = jnp.where(iota + 16 >= MAXSTEPS, nw_cnt, l1v)
        lobuf_v[pl.ds(32, 16)] = jnp.zeros((16,), jnp.int32) + nw_cnt

        p0 = posl_v[pl.ds(0, 16)][0]
        q0 = lidxl_v[pl.ds(0, 16)][0]

        def pad_body(t, _):
            gi = iota + t * 16
            sel = gi >= nw_cnt
            posl_v[pl.ds(t * 16, 16)] = jnp.where(
                sel, p0, posl_v[pl.ds(t * 16, 16)])
            lidxl_v[pl.ds(t * 16, 16)] = jnp.where(
                sel, q0, lidxl_v[pl.ds(t * 16, 16)])
            return 0
        lax.fori_loop(0, LISTCAP // 16, pad_body, 0)

        cnt_v[pl.ds(0, 16)] = jnp.zeros((16,), jnp.int32) + nw_cnt

        outs = [
            pltpu.make_async_copy(posl_v, posl_hbm.at[wid], sem_a),
            pltpu.make_async_copy(lidxl_v, lidxl_hbm.at[wid], sem_a),
            pltpu.make_async_copy(labl_v, labl_hbm.at[wid], sem_a),
            pltpu.make_async_copy(lobuf_v, lob_hbm.at[wid], sem_a),
            pltpu.make_async_copy(cnt_v, cnt_hbm.at[wid], sem_a),
        ]
        for o in outs:
            o.start()
        for o in outs:
            o.wait()

    return k(ind, newlab)


def _tc_norm(feature):
    RB = 2048

    def body(f_ref, out_ref):
        f = f_ref[...]
        out_ref[...] = f / (jnp.sqrt(jnp.sum(f * f, axis=1, keepdims=True)) + 1e-10)

    return pl.pallas_call(
        body,
        grid=(B // RB,),
        in_specs=[pl.BlockSpec((RB, D), lambda i: (i, 0))],
        out_specs=pl.BlockSpec((RB, D), lambda i: (i, 0)),
        out_shape=jax.ShapeDtypeStruct((B, D), jnp.float32),
    )(feature)


def _sc_update(bank, labels, fnorm, posl, lidxl, labl, lob, cnts):
    @functools.partial(
        pl.kernel,
        out_type=(
            jax.ShapeDtypeStruct((LENGTH, D), jnp.float32),
            jax.ShapeDtypeStruct((LENGTH,), jnp.int32),
        ),
        mesh=_mesh(),
        compiler_params=pltpu.CompilerParams(needs_layout_passes=False),
        scratch_types=[
            pltpu.VMEM((LISTCAP,), jnp.int32),
            pltpu.VMEM((LISTCAP,), jnp.int32),
            pltpu.VMEM((LISTCAP,), jnp.int32),
            pltpu.VMEM((48,), jnp.int32),
            pltpu.VMEM((16,), jnp.int32),
            pltpu.VMEM((SHARD,), jnp.int32),
            pltpu.VMEM((CC, D), jnp.float32),
            pltpu.VMEM((CC, D), jnp.float32),
            pltpu.VMEM((CC, D), jnp.float32),
            pltpu.VMEM((FNROWS, D), jnp.float32),
            pltpu.VMEM((FNROWS, D), jnp.float32),
            pltpu.SemaphoreType.DMA,
            pltpu.SemaphoreType.DMA,
            pltpu.SemaphoreType.DMA,
            pltpu.SemaphoreType.DMA,
            pltpu.SemaphoreType.DMA,
            pltpu.SemaphoreType.DMA,
            pltpu.SemaphoreType.DMA,
            pltpu.SemaphoreType.DMA,
            pltpu.SemaphoreType.DMA,
        ],
    )
    def k(bank_hbm, lab_hbm, fn_hbm, posl_hbm, lidxl_hbm, labl_hbm, lob_hbm,
          cnt_hbm, out_fb, out_lb,
          posl_v, lidxl_v, labl_v, lobuf_v, cnt_v, labsh_v,
          cb0, cb1, cb2, fb0, fb1,
          rs0, rs1, rs2, ws0, ws1, ws2, fs0, fs1, lsem):
        wid = lax.axis_index("s") * NC + lax.axis_index("c")
        base = wid * SHARD
        is_last = wid == NW - 1
        steps = jnp.where(is_last, LAST // CC, MAXSTEPS)
        cbufs, rsems, wsems = (cb0, cb1, cb2), (rs0, rs1, rs2), (ws0, ws1, ws2)
        fnbufs, fnsems = (fb0, fb1), (fs0, fs1)
        iota = _iota16()

        def rd_desc(i, b):
            return pltpu.make_async_copy(
                bank_hbm.at[pl.ds(base + i * CC, CC)], cbufs[b], rsems[b])

        def wr_desc(i, b):
            return pltpu.make_async_copy(
                cbufs[b], out_fb.at[pl.ds(base + i * CC, CC)], wsems[b])

        for i in range(2):
            @pl.when(jnp.int32(i) < steps)
            def _(i=i):
                rd_desc(i, i).start()

        ins = [
            pltpu.make_async_copy(posl_hbm.at[wid], posl_v, fs0),
            pltpu.make_async_copy(lidxl_hbm.at[wid], lidxl_v, fs0),
            pltpu.make_async_copy(labl_hbm.at[wid], labl_v, fs0),
            pltpu.make_async_copy(lob_hbm.at[wid], lobuf_v, fs0),
            pltpu.make_async_copy(cnt_hbm.at[wid], cnt_v, fs0),
        ]
        for o in ins:
            o.start()

        @pl.when(jnp.logical_not(is_last))
        def _():
            pltpu.make_async_copy(
                lab_hbm.at[pl.ds(base, SHARD)], labsh_v, fs1).start()

        @pl.when(is_last)
        def _():
            pltpu.make_async_copy(
                lab_hbm.at[pl.ds(base, LAST)],
                labsh_v.at[pl.ds(0, LAST)], fs1).start()

        for o in ins:
            o.wait()

        @pl.when(jnp.logical_not(is_last))
        def _():
            pltpu.make_async_copy(
                lab_hbm.at[pl.ds(base, SHARD)], labsh_v, fs1).wait()

        @pl.when(is_last)
        def _():
            pltpu.make_async_copy(
                lab_hbm.at[pl.ds(base, LAST)],
                labsh_v.at[pl.ds(0, LAST)], fs1).wait()

        nw_cnt = cnt_v[pl.ds(0, 16)][0]

        def region_lo(j):
            return plsc.load_gather(
                lobuf_v, [jnp.zeros((16,), jnp.int32) + j])[0]

        def region_lo8(j):
            return pl.multiple_of((region_lo(j) // 8) * 8, 8)

        def fn_n(j):
            return (region_lo(j + 1) - region_lo8(j) + FGR - 1) // FGR

        def fn_gather_start(j, f):
            lo8 = region_lo8(j)

            def sbody(g, _):
                off = pl.multiple_of(lo8 + g * FGR, 8)
                pltpu.make_async_copy(
                    fn_hbm.at[posl_v.at[pl.ds(off, FGR)]],
                    fnbufs[f].at[pl.ds(g * FGR, FGR)], fnsems[f]).start()
                return 0
            lax.fori_loop(0, fn_n(j), sbody, 0)

        def fn_gather_wait(j, f):
            lo8 = region_lo8(j)

            def wbody(g, _):
                pltpu.make_async_copy(
                    fn_hbm.at[posl_v.at[pl.ds(lo8, FGR)]],
                    fnbufs[f].at[pl.ds(0, FGR)], fnsems[f]).wait()
                return 0
            lax.fori_loop(0, fn_n(j), wbody, 0)

        with jax.named_scope("ph_labels"):
            def lab_body(j, _):
                m = (iota + j * 16) < nw_cnt
                lidx = jnp.where(m, lidxl_v[pl.ds(j * 16, 16)], 0)
                lv = labl_v[pl.ds(j * 16, 16)]
                plsc.store_scatter(labsh_v, [lidx], lv, mask=m)
                return 0
            lax.fori_loop(0, (nw_cnt + 15) // 16, lab_body, 0)

        @pl.when(jnp.logical_not(is_last))
        def _():
            pltpu.make_async_copy(
                labsh_v, out_lb.at[pl.ds(base, SHARD)], lsem).start()

        @pl.when(is_last)
        def _():
            pltpu.make_async_copy(
                labsh_v.at[pl.ds(0, LAST)],
                out_lb.at[pl.ds(base, LAST)], lsem).start()

        fn_gather_start(0, 0)

        def modify_region(i, b, f):
            lo = region_lo(i)
            lo8 = region_lo8(i)
            hie = region_lo(i + 1)
            cbuf = cbufs[b]
            fnbuf = fnbufs[f]

            iot = _iota16()

            def wbody(r, _):
                rsp = jnp.zeros((16,), jnp.int32) + r
                rowsel = plsc.load_gather(lidxl_v, [rsp]) - i * CC
                frsel = jnp.zeros((16,), jnp.int32) + (r - lo8)
                acc = jnp.zeros((16,), jnp.float32)
                nws = []
                for kk in range(D // 16):
                    cols = iot + kk * 16
                    o = plsc.load_gather(cbuf, [rowsel, cols])
                    fn = plsc.load_gather(fnbuf, [frsel, cols])
                    nwk = MOM * o + (1.0 - MOM) * fn
                    nws.append(nwk)
                    acc = acc + nwk * nwk
                s = jnp.sum(acc)
                sv = jnp.zeros((16,), jnp.float32) + s
                sq = sv * _rsqrt(sv)
                inv = 1.0 / (sq + 1e-10)
                for kk in range(D // 16):
                    cols = iot + kk * 16
                    plsc.store_scatter(cbuf, [rowsel, cols], nws[kk] * inv)
                return 0
            lax.fori_loop(lo, hie, wbody, 0)

        CYC = 6

        def ring_cycle(o, _):
            for kkk in range(CYC):
                i = o * CYC + kkk
                b = kkk % NBUF
                f = kkk % 2

                @pl.when(i < steps)
                def _(i=i, b=b, f=f):
                    rd_desc(i, b).wait()
                    fn_gather_wait(i, f)

                    @pl.when(i + 1 < steps)
                    def _():
                        fn_gather_start(i + 1, 1 - f)
                    modify_region(i, b, f)
                    wr_desc(i, b).start()

                @pl.when(i + 2 < steps)
                def _(i=i, b=b):
                    @pl.when(i >= 1)
                    def _():
                        wr_desc(i - 1, (b + NBUF - 1) % NBUF).wait()
                    rd_desc(i + 2, (b + 2) % NBUF).start()
            return 0

        with jax.named_scope("ph_ring"):
            lax.fori_loop(0, (MAXSTEPS + CYC - 1) // CYC, ring_cycle, 0)

            for b in range(NBUF):
                @pl.when(jnp.int32(b) < steps)
                def _(b=b):
                    wr_desc(0, b).wait()

        @pl.when(jnp.logical_not(is_last))
        def _():
            pltpu.make_async_copy(
                labsh_v, out_lb.at[pl.ds(base, SHARD)], lsem).wait()

        @pl.when(is_last)
        def _():
            pltpu.make_async_copy(
                labsh_v.at[pl.ds(0, LAST)],
                out_lb.at[pl.ds(base, LAST)], lsem).wait()

    return k(bank, labels, fnorm, posl, lidxl, labl, lob, cnts)


def kernel(feature_bank, label_bank, ind, feature, label):
    ind = ind.astype(jnp.int32)
    label = label.astype(jnp.int32)
    posl, lidxl, labl, lob, cnts = _sc_dedup(ind, label)
    fnorm = _tc_norm(feature)
    return _sc_update(feature_bank, label_bank, fnorm,
                      posl, lidxl, labl, lob, cnts)


# speedup vs baseline: 1.4813x; 1.0254x over previous
import functools

import jax
import jax.numpy as jnp
from jax import lax
from jax.experimental import pallas as pl
from jax.experimental.pallas import tpu as pltpu
from jax.experimental.pallas import tpu_sc as plsc

LENGTH = 100000
D = 128
B = 16384
MOM = 0.5

NC, NS, LANES = 2, 16, 16
NW = NC * NS
SHARD = 3200
LAST = LENGTH - (NW - 1) * SHARD
CC = 160
NBUF = 3
MAXSTEPS = SHARD // CC
TPR = CC // 16
FGR = 32
FNROWS = CC + FGR + 8
LISTCAP = SHARD + 256


def _mesh():
    return plsc.VectorSubcoreMesh(core_axis_name="c", subcore_axis_name="s")


def _iota16():
    return lax.iota(jnp.int32, 16)


def _rsqrt(x):
    xi = plsc.bitcast(x, jnp.int32)
    yi = jnp.int32(0x5F3759DF) - lax.shift_right_arithmetic(xi, 1)
    y = plsc.bitcast(yi, jnp.float32)
    for _ in range(3):
        y = y * (1.5 - 0.5 * x * y * y)
    return y


def _sc_dedup(ind, newlab):
    @functools.partial(
        pl.kernel,
        out_type=(
            jax.ShapeDtypeStruct((NW, LISTCAP), jnp.int32),
            jax.ShapeDtypeStruct((NW, LISTCAP), jnp.int32),
            jax.ShapeDtypeStruct((NW, LISTCAP), jnp.int32),
            jax.ShapeDtypeStruct((NW, 48), jnp.int32),
            jax.ShapeDtypeStruct((NW, 16), jnp.int32),
        ),
        mesh=_mesh(),
        compiler_params=pltpu.CompilerParams(needs_layout_passes=False),
        scratch_types=[
            pltpu.VMEM((B,), jnp.int32),
            pltpu.VMEM((B,), jnp.int32),
            pltpu.VMEM((SHARD,), jnp.int32),
            pltpu.VMEM((LISTCAP,), jnp.int32),
            pltpu.VMEM((LISTCAP,), jnp.int32),
            pltpu.VMEM((LISTCAP,), jnp.int32),
            pltpu.VMEM((48,), jnp.int32),
            pltpu.VMEM((16,), jnp.int32),
            pltpu.SemaphoreType.DMA,
            pltpu.SemaphoreType.DMA,
        ],
    )
    def k(ind_hbm, nlab_hbm,
          posl_hbm, lidxl_hbm, labl_hbm, lob_hbm, cnt_hbm,
          ind_v, labf_v, win_v, posl_v, lidxl_v, labl_v, lobuf_v, cnt_v,
          sem_a, sem_b):
        wid = lax.axis_index("s") * NC + lax.axis_index("c")
        base = wid * SHARD
        hi = jnp.minimum(base + SHARD, LENGTH)
        iota = _iota16()
        lane0 = iota == 0

        QB = B // 4
        ind_descs = [
            pltpu.make_async_copy(ind_hbm.at[pl.ds(c * QB, QB)],
                                  ind_v.at[pl.ds(c * QB, QB)], sem_a)
            for c in range(4)
        ]
        for dsc in ind_descs:
            dsc.start()
        d2 = pltpu.make_async_copy(nlab_hbm, labf_v, sem_b)
        d2.start()

        def zero_body(t, _):
            win_v[pl.ds(t * 16, 16)] = jnp.zeros((16,), jnp.int32)
            return 0
        lax.fori_loop(0, SHARD // 16, zero_body, 0)

        with jax.named_scope("ph_scan"):
            def scan_body(j, _):
                for u in range(2):
                    v = ind_v[pl.ds(j * 32 + u * 16, 16)]
                    pos1 = iota + (j * 32 + u * 16 + 1)
                    m = jnp.logical_and(v >= base, v < hi)
                    lidx = jnp.where(m, v - base, 0)
                    plsc.store_scatter(win_v, [lidx], pos1, mask=m)
                return 0
            for c in range(4):
                ind_descs[c].wait()
                lax.fori_loop(c * (QB // 32), (c + 1) * (QB // 32),
                              scan_body, 0)

        d2.wait()
        with jax.named_scope("ph_compact"):
            def cmp_body(t, off):
                @pl.when(t % TPR == 0)
                def _():
                    plsc.store_scatter(
                        lobuf_v, [jnp.zeros((16,), jnp.int32) + t // TPR],
                        jnp.zeros((16,), jnp.int32) + off, mask=lane0)
                wv = win_v[pl.ds(t * 16, 16)]
                m = wv > 0
                mi = m.astype(jnp.int32)
                pos = wv - 1
                tgt = off + plsc.cumsum(mi) - mi
                tgt = jnp.where(m, tgt, 0)
                plsc.store_scatter(posl_v, [tgt], pos, mask=m)
                lrow = iota + t * 16
                plsc.store_scatter(lidxl_v, [tgt], lrow, mask=m)
                lv = plsc.load_gather(labf_v, [jnp.where(m, pos, 0)], mask=m)
                plsc.store_scatter(labl_v, [tgt], lv, mask=m)
                return off + jnp.sum(mi)
            nw_cnt = lax.fori_loop(0, SHARD // 16, cmp_body, jnp.int32(0))

        l0v = lobuf_v[pl.ds(0, 16)]
        lobuf_v[pl.ds(0, 16)] = jnp.where(iota >= MAXSTEPS, nw_cnt, l0v)
        l1v = lobuf_v[pl.ds(16, 16)]
        lobuf_v[pl.ds(16, 16)] = jnp.where(iota + 16 >= MAXSTEPS, nw_cnt, l1v)
        lobuf_v[pl.ds(32, 16)] = jnp.zeros((16,), jnp.int32) + nw_cnt

        p0 = posl_v[pl.ds(0, 16)][0]
        nw8 = pl.multiple_of((nw_cnt // 8) * 8, 8)
        for q in range(3):
            off = pl.multiple_of(nw8 + q * 16, 8)
            v = posl_v[pl.ds(off, 16)]
            sel = (iota + off) >= nw_cnt
            posl_v[pl.ds(off, 16)] = jnp.where(sel, p0, v)

        cnt_v[pl.ds(0, 16)] = jnp.zeros((16,), jnp.int32) + nw_cnt

        outs = [
            pltpu.make_async_copy(posl_v, posl_hbm.at[wid], sem_a),
            pltpu.make_async_copy(lidxl_v, lidxl_hbm.at[wid], sem_a),
            pltpu.make_async_copy(labl_v, labl_hbm.at[wid], sem_a),
            pltpu.make_async_copy(lobuf_v, lob_hbm.at[wid], sem_a),
            pltpu.make_async_copy(cnt_v, cnt_hbm.at[wid], sem_a),
        ]
        for o in outs:
            o.start()
        for o in outs:
            o.wait()

    return k(ind, newlab)


def _tc_norm(feature):
    RB = 2048

    def body(f_ref, out_ref):
        f = f_ref[...]
        out_ref[...] = f / (jnp.sqrt(jnp.sum(f * f, axis=1, keepdims=True)) + 1e-10)

    return pl.pallas_call(
        body,
        grid=(B // RB,),
        in_specs=[pl.BlockSpec((RB, D), lambda i: (i, 0))],
        out_specs=pl.BlockSpec((RB, D), lambda i: (i, 0)),
        out_shape=jax.ShapeDtypeStruct((B, D), jnp.float32),
    )(feature)


def _sc_update(bank, labels, fnorm, posl, lidxl, labl, lob, cnts):
    @functools.partial(
        pl.kernel,
        out_type=(
            jax.ShapeDtypeStruct((LENGTH, D), jnp.float32),
            jax.ShapeDtypeStruct((LENGTH,), jnp.int32),
        ),
        mesh=_mesh(),
        compiler_params=pltpu.CompilerParams(needs_layout_passes=False),
        scratch_types=[
            pltpu.VMEM((LISTCAP,), jnp.int32),
            pltpu.VMEM((LISTCAP,), jnp.int32),
            pltpu.VMEM((LISTCAP,), jnp.int32),
            pltpu.VMEM((48,), jnp.int32),
            pltpu.VMEM((16,), jnp.int32),
            pltpu.VMEM((SHARD,), jnp.int32),
            pltpu.VMEM((CC, D), jnp.float32),
            pltpu.VMEM((CC, D), jnp.float32),
            pltpu.VMEM((CC, D), jnp.float32),
            pltpu.VMEM((FNROWS, D), jnp.float32),
            pltpu.VMEM((FNROWS, D), jnp.float32),
            pltpu.SemaphoreType.DMA,
            pltpu.SemaphoreType.DMA,
            pltpu.SemaphoreType.DMA,
            pltpu.SemaphoreType.DMA,
            pltpu.SemaphoreType.DMA,
            pltpu.SemaphoreType.DMA,
            pltpu.SemaphoreType.DMA,
            pltpu.SemaphoreType.DMA,
            pltpu.SemaphoreType.DMA,
        ],
    )
    def k(bank_hbm, lab_hbm, fn_hbm, posl_hbm, lidxl_hbm, labl_hbm, lob_hbm,
          cnt_hbm, out_fb, out_lb,
          posl_v, lidxl_v, labl_v, lobuf_v, cnt_v, labsh_v,
          cb0, cb1, cb2, fb0, fb1,
          rs0, rs1, rs2, ws0, ws1, ws2, fs0, fs1, lsem):
        wid = lax.axis_index("s") * NC + lax.axis_index("c")
        base = wid * SHARD
        is_last = wid == NW - 1
        steps = jnp.where(is_last, LAST // CC, MAXSTEPS)
        cbufs, rsems, wsems = (cb0, cb1, cb2), (rs0, rs1, rs2), (ws0, ws1, ws2)
        fnbufs, fnsems = (fb0, fb1), (fs0, fs1)
        iota = _iota16()

        def rd_desc(i, b):
            return pltpu.make_async_copy(
                bank_hbm.at[pl.ds(base + i * CC, CC)], cbufs[b], rsems[b])

        def wr_desc(i, b):
            return pltpu.make_async_copy(
                cbufs[b], out_fb.at[pl.ds(base + i * CC, CC)], wsems[b])

        for i in range(2):
            @pl.when(jnp.int32(i) < steps)
            def _(i=i):
                rd_desc(i, i).start()

        ins = [
            pltpu.make_async_copy(posl_hbm.at[wid], posl_v, fs0),
            pltpu.make_async_copy(lidxl_hbm.at[wid], lidxl_v, fs0),
            pltpu.make_async_copy(labl_hbm.at[wid], labl_v, fs0),
            pltpu.make_async_copy(lob_hbm.at[wid], lobuf_v, fs0),
            pltpu.make_async_copy(cnt_hbm.at[wid], cnt_v, fs0),
        ]
        for o in ins:
            o.start()

        @pl.when(jnp.logical_not(is_last))
        def _():
            pltpu.make_async_copy(
                lab_hbm.at[pl.ds(base, SHARD)], labsh_v, fs1).start()

        @pl.when(is_last)
        def _():
            pltpu.make_async_copy(
                lab_hbm.at[pl.ds(base, LAST)],
                labsh_v.at[pl.ds(0, LAST)], fs1).start()

        for o in ins:
            o.wait()

        @pl.when(jnp.logical_not(is_last))
        def _():
            pltpu.make_async_copy(
                lab_hbm.at[pl.ds(base, SHARD)], labsh_v, fs1).wait()

        @pl.when(is_last)
        def _():
            pltpu.make_async_copy(
                lab_hbm.at[pl.ds(base, LAST)],
                labsh_v.at[pl.ds(0, LAST)], fs1).wait()

        nw_cnt = cnt_v[pl.ds(0, 16)][0]

        def region_lo(j):
            return plsc.load_gather(
                lobuf_v, [jnp.zeros((16,), jnp.int32) + j])[0]

        def region_lo8(j):
            return pl.multiple_of((region_lo(j) // 8) * 8, 8)

        def fn_n(j):
            return (region_lo(j + 1) - region_lo8(j) + FGR - 1) // FGR

        def fn_gather_start(j, f):
            lo8 = region_lo8(j)

            def sbody(g, _):
                off = pl.multiple_of(lo8 + g * FGR, 8)
                pltpu.make_async_copy(
                    fn_hbm.at[posl_v.at[pl.ds(off, FGR)]],
                    fnbufs[f].at[pl.ds(g * FGR, FGR)], fnsems[f]).start()
                return 0
            lax.fori_loop(0, fn_n(j), sbody, 0)

        def fn_gather_wait(j, f):
            lo8 = region_lo8(j)

            def wbody(g, _):
                pltpu.make_async_copy(
                    fn_hbm.at[posl_v.at[pl.ds(lo8, FGR)]],
                    fnbufs[f].at[pl.ds(0, FGR)], fnsems[f]).wait()
                return 0
            lax.fori_loop(0, fn_n(j), wbody, 0)

        with jax.named_scope("ph_labels"):
            def lab_body(j, _):
                m = (iota + j * 16) < nw_cnt
                lidx = jnp.where(m, lidxl_v[pl.ds(j * 16, 16)], 0)
                lv = labl_v[pl.ds(j * 16, 16)]
                plsc.store_scatter(labsh_v, [lidx], lv, mask=m)
                return 0
            lax.fori_loop(0, (nw_cnt + 15) // 16, lab_body, 0)

        @pl.when(jnp.logical_not(is_last))
        def _():
            pltpu.make_async_copy(
                labsh_v, out_lb.at[pl.ds(base, SHARD)], lsem).start()

        @pl.when(is_last)
        def _():
            pltpu.make_async_copy(
                labsh_v.at[pl.ds(0, LAST)],
                out_lb.at[pl.ds(base, LAST)], lsem).start()

        fn_gather_start(0, 0)

        def modify_region(i, b, f):
            lo = region_lo(i)
            lo8 = region_lo8(i)
            hie = region_lo(i + 1)
            cbuf = cbufs[b]
            fnbuf = fnbufs[f]

            iot = _iota16()

            def wbody(r, _):
                rsp = jnp.zeros((16,), jnp.int32) + r
                rowsel = plsc.load_gather(lidxl_v, [rsp]) - i * CC
                frsel = jnp.zeros((16,), jnp.int32) + (r - lo8)
                acc = jnp.zeros((16,), jnp.float32)
                nws = []
                for kk in range(D // 16):
                    cols = iot + kk * 16
                    o = plsc.load_gather(cbuf, [rowsel, cols])
                    fn = plsc.load_gather(fnbuf, [frsel, cols])
                    nwk = MOM * o + (1.0 - MOM) * fn
                    nws.append(nwk)
                    acc = acc + nwk * nwk
                s = jnp.sum(acc)
                sv = jnp.zeros((16,), jnp.float32) + s
                sq = sv * _rsqrt(sv)
                inv = 1.0 / (sq + 1e-10)
                for kk in range(D // 16):
                    cols = iot + kk * 16
                    plsc.store_scatter(cbuf, [rowsel, cols], nws[kk] * inv)
                return 0
            lax.fori_loop(lo, hie, wbody, 0)

        CYC = 6

        def ring_cycle(o, _):
            for kkk in range(CYC):
                i = o * CYC + kkk
                b = kkk % NBUF
                f = kkk % 2

                @pl.when(i < steps)
                def _(i=i, b=b, f=f):
                    rd_desc(i, b).wait()
                    fn_gather_wait(i, f)

                    @pl.when(i + 1 < steps)
                    def _():
                        fn_gather_start(i + 1, 1 - f)
                    modify_region(i, b, f)
                    wr_desc(i, b).start()

                @pl.when(i + 2 < steps)
                def _(i=i, b=b):
                    @pl.when(i >= 1)
                    def _():
                        wr_desc(i - 1, (b + NBUF - 1) % NBUF).wait()
                    rd_desc(i + 2, (b + 2) % NBUF).start()
            return 0

        with jax.named_scope("ph_ring"):
            lax.fori_loop(0, (MAXSTEPS + CYC - 1) // CYC, ring_cycle, 0)

            for b in range(NBUF):
                @pl.when(jnp.int32(b) < steps)
                def _(b=b):
                    wr_desc(0, b).wait()

        @pl.when(jnp.logical_not(is_last))
        def _():
            pltpu.make_async_copy(
                labsh_v, out_lb.at[pl.ds(base, SHARD)], lsem).wait()

        @pl.when(is_last)
        def _():
            pltpu.make_async_copy(
                labsh_v.at[pl.ds(0, LAST)],
                out_lb.at[pl.ds(base, LAST)], lsem).wait()

    return k(bank, labels, fnorm, posl, lidxl, labl, lob, cnts)


def kernel(feature_bank, label_bank, ind, feature, label):
    ind = ind.astype(jnp.int32)
    label = label.astype(jnp.int32)
    posl, lidxl, labl, lob, cnts = _sc_dedup(ind, label)
    fnorm = _tc_norm(feature)
    return _sc_update(feature_bank, label_bank, fnorm,
                      posl, lidxl, labl, lob, cnts)
